# Initial kernel scaffold; baseline (speedup 1.0000x reference)
import jax
import jax.numpy as jnp
from jax.experimental import pallas as pl


def kernel(x, pos, edge_attr, W1, b1, R1a, R1b, Wm0_1, Wm1_1, Ws1, W2, b2, R2a, R2b, Wm0_2, Wm1_2, Ws2, Wp0, Wp1, edge_index, batch):
    raise NotImplementedError("write your pallas kernel here")


# speedup vs baseline: 2.1414x; 2.1414x over previous
import functools

import jax
import jax.numpy as jnp
import numpy as np
from jax import lax
from jax.experimental import pallas as pl
from jax.experimental.pallas import tpu as pltpu
from jax.experimental.pallas import tpu_sc as plsc

_N = 10000
_E = 320000
_C = 128
_CB = 32
_NBLK = 4
_K = 128
_NCHUNK = _E // _K
_TILES = 16
_NPAD = 10240
_ROWS_PER_TILE = _NPAD // _TILES
_C1 = float(np.sqrt(3.0 / (4.0 * np.pi)))

_NB_NODE = 2000
_EB = 8000


def _dot_t(a, bt):
    return lax.dot_general(a, bt, (((1,), (1,)), ((), ())),
                           preferred_element_type=jnp.float32)


def _feats_tc(x, W, b):
    da = x.shape[1]
    wt = W.T.reshape(_NBLK, _CB, da)

    def body(x_ref, w_ref, b_ref, o_ref):
        j = pl.program_id(1)
        val = _dot_t(x_ref[...], w_ref[0])
        o_ref[...] = (val + b_ref[pl.ds(j, 1), :])[None]

    return pl.pallas_call(
        body,
        grid=(_N // _NB_NODE, _NBLK),
        in_specs=[
            pl.BlockSpec((_NB_NODE, da), lambda i, j: (i, 0)),
            pl.BlockSpec((1, _CB, da), lambda i, j: (j, 0, 0)),
            pl.BlockSpec((_NBLK, _CB), lambda i, j: (0, 0)),
        ],
        out_specs=pl.BlockSpec((1, _NB_NODE, _CB), lambda i, j: (j, i, 0)),
        out_shape=jax.ShapeDtypeStruct((_NBLK, _N, _CB), jnp.float32),
    )(x, wt, b.reshape(_NBLK, _CB))


def _radial_tc(ea, Ra, Rbp):
    rbt = Rbp.T.reshape(_NBLK, 64, 64)

    def body(e_ref, ra_ref, rb_ref, o_ref):
        h = jax.nn.silu(
            jnp.dot(e_ref[...], ra_ref[...], preferred_element_type=jnp.float32))
        o_ref[...] = _dot_t(h, rb_ref[0])[None]

    return pl.pallas_call(
        body,
        grid=(_E // _EB, _NBLK),
        in_specs=[
            pl.BlockSpec((_EB, 16), lambda i, j: (i, 0)),
            pl.BlockSpec((16, 64), lambda i, j: (0, 0)),
            pl.BlockSpec((1, 64, 64), lambda i, j: (j, 0, 0)),
        ],
        out_specs=pl.BlockSpec((1, _EB, 64), lambda i, j: (j, i, 0)),
        out_shape=jax.ShapeDtypeStruct((_NBLK, _E, 64), jnp.float32),
    )(ea, Ra, rbt)


def _mace_node_parts(b_ref):
    bv = b_ref[...]
    m0 = jnp.concatenate([bv[k, :, 0:32] for k in range(_NBLK)], axis=-1)
    m1x = jnp.concatenate([bv[k, :, 32:64] for k in range(_NBLK)], axis=-1)
    m1y = jnp.concatenate([bv[k, :, 64:96] for k in range(_NBLK)], axis=-1)
    m1z = jnp.concatenate([bv[k, :, 96:128] for k in range(_NBLK)], axis=-1)
    return m0, m1x, m1y, m1z


def _node_tc(bacc, attrs, Wm0, Wm1, Ws):
    da = attrs.shape[1]

    def body(b_ref, a_ref, wm0_ref, wm1_ref, ws_ref, o_ref):
        m0, m1x, m1y, m1z = _mace_node_parts(b_ref)
        s = m0 + m0 * m0 + m1x * m1x + m1y * m1y + m1z * m1z
        h0 = (jnp.dot(s, wm0_ref[...], preferred_element_type=jnp.float32)
              + jnp.dot(a_ref[...], ws_ref[...], preferred_element_type=jnp.float32))
        one_b0 = 1.0 + m0
        h1x = jnp.dot(m1x * one_b0, wm1_ref[...], preferred_element_type=jnp.float32)
        h1y = jnp.dot(m1y * one_b0, wm1_ref[...], preferred_element_type=jnp.float32)
        h1z = jnp.dot(m1z * one_b0, wm1_ref[...], preferred_element_type=jnp.float32)
        o_ref[...] = jnp.concatenate([h0, h1x, h1y, h1z], axis=-1)

    return pl.pallas_call(
        body,
        grid=(_N // _NB_NODE,),
        in_specs=[
            pl.BlockSpec((_NBLK, _NB_NODE, _C), lambda i: (0, i, 0)),
            pl.BlockSpec((_NB_NODE, da), lambda i: (i, 0)),
            pl.BlockSpec((_C, _C), lambda i: (0, 0)),
            pl.BlockSpec((_C, _C), lambda i: (0, 0)),
            pl.BlockSpec((da, _C), lambda i: (0, 0)),
        ],
        out_specs=pl.BlockSpec((_NB_NODE, 4 * _C), lambda i: (i, 0)),
        out_shape=jax.ShapeDtypeStruct((_N, 4 * _C), jnp.float32),
    )(bacc, attrs, Wm0, Wm1, Ws)


def _node_final_tc(bacc, attrs, Wm0, Wm1, Ws, Wp0, Wp1, batch2d):
    da = attrs.shape[1]

    def body(b_ref, a_ref, wm0_ref, wm1_ref, ws_ref, wp0_ref, wp1_ref,
             bat_ref, sum_ref, cnt_ref):
        i = pl.program_id(0)
        m0, m1x, m1y, m1z = _mace_node_parts(b_ref)
        s = m0 + m0 * m0 + m1x * m1x + m1y * m1y + m1z * m1z
        h0 = (jnp.dot(s, wm0_ref[...], preferred_element_type=jnp.float32)
              + jnp.dot(a_ref[...], ws_ref[...], preferred_element_type=jnp.float32))
        one_b0 = 1.0 + m0
        h1x = jnp.dot(m1x * one_b0, wm1_ref[...], preferred_element_type=jnp.float32)
        h1y = jnp.dot(m1y * one_b0, wm1_ref[...], preferred_element_type=jnp.float32)
        h1z = jnp.dot(m1z * one_b0, wm1_ref[...], preferred_element_type=jnp.float32)
        pr = jnp.sum(jnp.dot(h0, wp0_ref[...], preferred_element_type=jnp.float32) * h0,
                     axis=1, keepdims=True)
        for h1 in (h1x, h1y, h1z):
            pr = pr + jnp.sum(
                jnp.dot(h1, wp1_ref[...], preferred_element_type=jnp.float32) * h1,
                axis=1, keepdims=True)
        bat = bat_ref[...]

        @pl.when(i == 0)
        def _():
            sum_ref[...] = jnp.zeros((8, 1), jnp.float32)
            cnt_ref[...] = jnp.zeros((8, 1), jnp.float32)

        for g in range(8):
            m = bat == g
            sum_ref[g:g + 1, :] += jnp.sum(jnp.where(m, pr, 0.0), axis=0, keepdims=True)
            cnt_ref[g:g + 1, :] += jnp.sum(jnp.where(m, 1.0, 0.0), axis=0, keepdims=True)

    return pl.pallas_call(
        body,
        grid=(_N // _NB_NODE,),
        in_specs=[
            pl.BlockSpec((_NBLK, _NB_NODE, _C), lambda i: (0, i, 0)),
            pl.BlockSpec((_NB_NODE, da), lambda i: (i, 0)),
            pl.BlockSpec((_C, _C), lambda i: (0, 0)),
            pl.BlockSpec((_C, _C), lambda i: (0, 0)),
            pl.BlockSpec((da, _C), lambda i: (0, 0)),
            pl.BlockSpec((_C, _C), lambda i: (0, 0)),
            pl.BlockSpec((_C, _C), lambda i: (0, 0)),
            pl.BlockSpec((_NB_NODE, 1), lambda i: (i, 0)),
        ],
        out_specs=[
            pl.BlockSpec((8, 1), lambda i: (0, 0)),
            pl.BlockSpec((8, 1), lambda i: (0, 0)),
        ],
        out_shape=[
            jax.ShapeDtypeStruct((8, 1), jnp.float32),
            jax.ShapeDtypeStruct((8, 1), jnp.float32),
        ],
    )(bacc, attrs, Wm0, Wm1, Ws, Wp0, Wp1, batch2d)


def _sc_params():
    return pltpu.CompilerParams(needs_layout_passes=False,
                                use_tc_tiling_on_sc=False)


def _sc_mesh():
    return plsc.VectorSubcoreMesh(core_axis_name="c", subcore_axis_name="s")


def _unit_y1_16(px, py, pz, sv, dv):
    dx = plsc.load_gather(px, [dv]) - plsc.load_gather(px, [sv])
    dy = plsc.load_gather(py, [dv]) - plsc.load_gather(py, [sv])
    dz = plsc.load_gather(pz, [dv]) - plsc.load_gather(pz, [sv])
    r2 = dx * dx + dy * dy + dz * dz
    ii = plsc.bitcast(r2, jnp.int32)
    ii = 0x5F3759DF - jnp.right_shift(ii, 1)
    y = plsc.bitcast(ii, jnp.float32)
    for _ in range(3):
        y = y * (1.5 - 0.5 * r2 * y * y)
    r = r2 * y
    rc = _C1 / (r + 1e-9)
    return dx * rc, dy * rc, dz * rc


def _sc_y1(posx, posy, posz, src, dst):
    @functools.partial(
        pl.kernel,
        out_type=[jax.ShapeDtypeStruct((_E,), jnp.float32)] * 3,
        mesh=_sc_mesh(),
        compiler_params=_sc_params(),
        scratch_types=[
            pltpu.VMEM((_NPAD,), jnp.float32),
            pltpu.VMEM((_NPAD,), jnp.float32),
            pltpu.VMEM((_NPAD,), jnp.float32),
            pltpu.VMEM((_K,), jnp.int32),
            pltpu.VMEM((_K,), jnp.int32),
            pltpu.VMEM((_K,), jnp.float32),
            pltpu.VMEM((_K,), jnp.float32),
            pltpu.VMEM((_K,), jnp.float32),
        ],
    )
    def k(posx_h, posy_h, posz_h, src_h, dst_h, ox_h, oy_h, oz_h,
          px, py, pz, src_v, dst_v, yx_v, yy_v, yz_v):
        c = lax.axis_index("c")
        s = lax.axis_index("s")
        wid = c * _TILES + s
        pltpu.sync_copy(posx_h, px)
        pltpu.sync_copy(posy_h, py)
        pltpu.sync_copy(posz_h, pz)
        nchunks = (_NCHUNK - wid + 2 * _TILES - 1) // (2 * _TILES)

        def chunk(i, carry):
            e0 = (wid + 2 * _TILES * i) * _K
            pltpu.sync_copy(src_h.at[pl.ds(e0, _K)], src_v)
            pltpu.sync_copy(dst_h.at[pl.ds(e0, _K)], dst_v)
            for kk in range(8):
                sl = pl.ds(16 * kk, 16)
                yx, yy, yz = _unit_y1_16(px, py, pz, src_v[sl], dst_v[sl])
                yx_v[sl] = yx
                yy_v[sl] = yy
                yz_v[sl] = yz
            pltpu.sync_copy(yx_v, ox_h.at[pl.ds(e0, _K)])
            pltpu.sync_copy(yy_v, oy_h.at[pl.ds(e0, _K)])
            pltpu.sync_copy(yz_v, oz_h.at[pl.ds(e0, _K)])
            return carry

        lax.fori_loop(0, nchunks, chunk, 0)

    return k(posx, posy, posz, src, dst)


def _sc_edge(y1x, y1y, y1z, src, dst, nf4, w4):
    @functools.partial(
        pl.kernel,
        out_type=jax.ShapeDtypeStruct((_NBLK * _NPAD, _C), jnp.float32),
        mesh=_sc_mesh(),
        compiler_params=_sc_params(),
        scratch_types=[
            pltpu.VMEM((_K,), jnp.int32),
            pltpu.VMEM((_K,), jnp.int32),
            pltpu.VMEM((_K,), jnp.int32),
            pltpu.VMEM((_K, _CB), jnp.float32),
            pltpu.VMEM((_K, 64), jnp.float32),
            pltpu.VMEM((_K,), jnp.float32),
            pltpu.VMEM((_K,), jnp.float32),
            pltpu.VMEM((_K,), jnp.float32),
            pltpu.VMEM((_K, _C), jnp.float32),
            pltpu.VMEM_SHARED((_NPAD, _C), jnp.float32),
            pltpu.SemaphoreType.DMA,
        ],
    )
    def k(y1x_h, y1y_h, y1z_h, src_h, dst_h, nf_h, w_h, out_h,
          src_v, dst_v, idx2, fs, wv, y1x_v, y1y_v, y1z_v, out_v,
          acc, sem):
        c = lax.axis_index("c")
        s = lax.axis_index("s")

        for p in range(2):
            b = 2 * c + p

            def zero_row(r, carry):
                for kk in range(8):
                    out_v[r, pl.ds(16 * kk, 16)] = jnp.zeros((16,), jnp.float32)
                return carry

            lax.fori_loop(0, _K, zero_row, 0)
            for t in range(5):
                row0 = s * _ROWS_PER_TILE + t * _K
                pltpu.sync_copy(out_v.at[pl.ds(0, _K), :],
                                acc.at[pl.ds(row0, _K), :])
            plsc.subcore_barrier()

            nchunks = (_NCHUNK - s + _TILES - 1) // _TILES

            def chunk(i, carry):
                e0 = (s + _TILES * i) * _K
                pltpu.sync_copy(src_h.at[pl.ds(e0, _K)], src_v)
                pltpu.sync_copy(dst_h.at[pl.ds(e0, _K)], dst_v)
                pltpu.sync_copy(w_h.at[pl.ds(b * _E + e0, _K), :], wv)
                pltpu.sync_copy(y1x_h.at[pl.ds(e0, _K)], y1x_v)
                pltpu.sync_copy(y1y_h.at[pl.ds(e0, _K)], y1y_v)
                pltpu.sync_copy(y1z_h.at[pl.ds(e0, _K)], y1z_v)
                bn = b * _NPAD
                for kk in range(8):
                    sl = pl.ds(16 * kk, 16)
                    idx2[sl] = src_v[sl] + bn
                pltpu.async_copy(nf_h.at[idx2], fs, sem).wait()

                def edge(j, carry2):
                    f0 = fs[j, pl.ds(0, 16)]
                    f1 = fs[j, pl.ds(16, 16)]
                    w00 = wv[j, pl.ds(0, 16)]
                    w01 = wv[j, pl.ds(16, 16)]
                    w10 = wv[j, pl.ds(32, 16)]
                    w11 = wv[j, pl.ds(48, 16)]
                    jj = jnp.full((16,), j, jnp.int32)
                    yx = plsc.load_gather(y1x_v, [jj])
                    yy = plsc.load_gather(y1y_v, [jj])
                    yz = plsc.load_gather(y1z_v, [jj])
                    t0 = f0 * w10
                    t1 = f1 * w11
                    out_v[j, pl.ds(0, 16)] = f0 * w00
                    out_v[j, pl.ds(16, 16)] = f1 * w01
                    out_v[j, pl.ds(32, 16)] = t0 * yx
                    out_v[j, pl.ds(48, 16)] = t1 * yx
                    out_v[j, pl.ds(64, 16)] = t0 * yy
                    out_v[j, pl.ds(80, 16)] = t1 * yy
                    out_v[j, pl.ds(96, 16)] = t0 * yz
                    out_v[j, pl.ds(112, 16)] = t1 * yz
                    return carry2

                lax.fori_loop(0, _K, edge, 0)
                pltpu.sync_copy(out_v, acc.at[dst_v], add=True)
                return carry

            lax.fori_loop(0, nchunks, chunk, 0)
            plsc.subcore_barrier()
            for t in range(5):
                row0 = s * _ROWS_PER_TILE + t * _K
                pltpu.sync_copy(acc.at[pl.ds(row0, _K), :],
                                out_h.at[pl.ds(b * _NPAD + row0, _K), :])

    return k(y1x, y1y, y1z, src, dst, nf4, w4)


def _pad_feats(f):
    return jnp.pad(f, ((0, 0), (0, _NPAD - _N), (0, 0))).reshape(_NBLK * _NPAD, _CB)


def _perms():
    colperm = np.concatenate([
        np.concatenate([np.arange(32 * b, 32 * b + 32),
                        128 + np.arange(32 * b, 32 * b + 32)])
        for b in range(_NBLK)])
    rowperm = np.concatenate([
        np.arange(128),
        np.array([128 + 3 * cc + ax for ax in range(3) for cc in range(128)])])
    return colperm, rowperm


def kernel(x, pos, edge_attr, W1, b1, R1a, R1b, Wm0_1, Wm1_1, Ws1,
           W2, b2, R2a, R2b, Wm0_2, Wm1_2, Ws2, Wp0, Wp1, edge_index, batch):
    colperm, rowperm = _perms()
    src = edge_index[0]
    dst = edge_index[1]
    pos_p = jnp.pad(pos, ((0, _NPAD - _N), (0, 0)))
    posx = pos_p[:, 0]
    posy = pos_p[:, 1]
    posz = pos_p[:, 2]

    y1x, y1y, y1z = _sc_y1(posx, posy, posz, src, dst)

    f1 = _pad_feats(_feats_tc(x, W1, b1))
    r1 = _radial_tc(edge_attr, R1a, R1b[:, colperm]).reshape(_NBLK * _E, 64)
    bacc1 = _sc_edge(y1x, y1y, y1z, src, dst, f1, r1)
    bacc1 = bacc1.reshape(_NBLK, _NPAD, _C)[:, :_N, :]
    u1 = _node_tc(bacc1, x, Wm0_1, Wm1_1, Ws1)

    f2 = _pad_feats(_feats_tc(u1, W2[rowperm], b2))
    r2 = _radial_tc(edge_attr, R2a, R2b[:, colperm]).reshape(_NBLK * _E, 64)
    bacc2 = _sc_edge(y1x, y1y, y1z, src, dst, f2, r2)
    bacc2 = bacc2.reshape(_NBLK, _NPAD, _C)[:, :_N, :]
    sums, counts = _node_final_tc(bacc2, u1,
                                  Wm0_2, Wm1_2, Ws2[rowperm], Wp0, Wp1,
                                  batch.reshape(_N, 1).astype(jnp.int32))
    return sums / jnp.maximum(counts, 1.0)


# speedup vs baseline: 3.3802x; 1.5785x over previous
import functools

import jax
import jax.numpy as jnp
import numpy as np
from jax import lax
from jax.experimental import pallas as pl
from jax.experimental.pallas import tpu as pltpu
from jax.experimental.pallas import tpu_sc as plsc

_N = 10000
_E = 320000
_C = 128
_CB = 32
_NBLK = 4
_K = 128
_NCHUNK = _E // _K
_TILES = 16
_NPAD = 10240
_ROWS_PER_TILE = _NPAD // _TILES
_C1 = float(np.sqrt(3.0 / (4.0 * np.pi)))

_NB_NODE = 2000
_EB = 8000


def _dot_t(a, bt):
    return lax.dot_general(a, bt, (((1,), (1,)), ((), ())),
                           preferred_element_type=jnp.float32)


def _feats_tc(x, W, b):
    da = x.shape[1]
    wt = W.T.reshape(_NBLK, _CB, da)

    def body(x_ref, w_ref, b_ref, o_ref):
        j = pl.program_id(1)
        val = _dot_t(x_ref[...], w_ref[0])
        o_ref[...] = (val + b_ref[pl.ds(j, 1), :])[None]

    return pl.pallas_call(
        body,
        grid=(_N // _NB_NODE, _NBLK),
        in_specs=[
            pl.BlockSpec((_NB_NODE, da), lambda i, j: (i, 0)),
            pl.BlockSpec((1, _CB, da), lambda i, j: (j, 0, 0)),
            pl.BlockSpec((_NBLK, _CB), lambda i, j: (0, 0)),
        ],
        out_specs=pl.BlockSpec((1, _NB_NODE, _CB), lambda i, j: (j, i, 0)),
        out_shape=jax.ShapeDtypeStruct((_NBLK, _N, _CB), jnp.float32),
    )(x, wt, b.reshape(_NBLK, _CB))


def _radial_tc(ea, Ra, Rbp):
    rbt = Rbp.T.reshape(_NBLK, 64, 64)

    def body(e_ref, ra_ref, rb_ref, o_ref):
        h = jax.nn.silu(
            jnp.dot(e_ref[...], ra_ref[...], preferred_element_type=jnp.float32))
        o_ref[...] = _dot_t(h, rb_ref[0])[None]

    return pl.pallas_call(
        body,
        grid=(_E // _EB, _NBLK),
        in_specs=[
            pl.BlockSpec((_EB, 16), lambda i, j: (i, 0)),
            pl.BlockSpec((16, 64), lambda i, j: (0, 0)),
            pl.BlockSpec((1, 64, 64), lambda i, j: (j, 0, 0)),
        ],
        out_specs=pl.BlockSpec((1, _EB, 64), lambda i, j: (j, i, 0)),
        out_shape=jax.ShapeDtypeStruct((_NBLK, _E, 64), jnp.float32),
    )(ea, Ra, rbt)


def _mace_node_parts(b_ref):
    bv = b_ref[...]
    m0 = jnp.concatenate([bv[k, :, 0:32] for k in range(_NBLK)], axis=-1)
    m1x = jnp.concatenate([bv[k, :, 32:64] for k in range(_NBLK)], axis=-1)
    m1y = jnp.concatenate([bv[k, :, 64:96] for k in range(_NBLK)], axis=-1)
    m1z = jnp.concatenate([bv[k, :, 96:128] for k in range(_NBLK)], axis=-1)
    return m0, m1x, m1y, m1z


def _node_tc(bacc, attrs, Wm0, Wm1, Ws):
    da = attrs.shape[1]

    def body(b_ref, a_ref, wm0_ref, wm1_ref, ws_ref, o_ref):
        m0, m1x, m1y, m1z = _mace_node_parts(b_ref)
        s = m0 + m0 * m0 + m1x * m1x + m1y * m1y + m1z * m1z
        h0 = (jnp.dot(s, wm0_ref[...], preferred_element_type=jnp.float32)
              + jnp.dot(a_ref[...], ws_ref[...], preferred_element_type=jnp.float32))
        one_b0 = 1.0 + m0
        h1x = jnp.dot(m1x * one_b0, wm1_ref[...], preferred_element_type=jnp.float32)
        h1y = jnp.dot(m1y * one_b0, wm1_ref[...], preferred_element_type=jnp.float32)
        h1z = jnp.dot(m1z * one_b0, wm1_ref[...], preferred_element_type=jnp.float32)
        o_ref[...] = jnp.concatenate([h0, h1x, h1y, h1z], axis=-1)

    return pl.pallas_call(
        body,
        grid=(_N // _NB_NODE,),
        in_specs=[
            pl.BlockSpec((_NBLK, _NB_NODE, _C), lambda i: (0, i, 0)),
            pl.BlockSpec((_NB_NODE, da), lambda i: (i, 0)),
            pl.BlockSpec((_C, _C), lambda i: (0, 0)),
            pl.BlockSpec((_C, _C), lambda i: (0, 0)),
            pl.BlockSpec((da, _C), lambda i: (0, 0)),
        ],
        out_specs=pl.BlockSpec((_NB_NODE, 4 * _C), lambda i: (i, 0)),
        out_shape=jax.ShapeDtypeStruct((_N, 4 * _C), jnp.float32),
    )(bacc, attrs, Wm0, Wm1, Ws)


def _node_final_tc(bacc, attrs, Wm0, Wm1, Ws, Wp0, Wp1, batch2d):
    da = attrs.shape[1]

    def body(b_ref, a_ref, wm0_ref, wm1_ref, ws_ref, wp0_ref, wp1_ref,
             bat_ref, sum_ref, cnt_ref):
        i = pl.program_id(0)
        m0, m1x, m1y, m1z = _mace_node_parts(b_ref)
        s = m0 + m0 * m0 + m1x * m1x + m1y * m1y + m1z * m1z
        h0 = (jnp.dot(s, wm0_ref[...], preferred_element_type=jnp.float32)
              + jnp.dot(a_ref[...], ws_ref[...], preferred_element_type=jnp.float32))
        one_b0 = 1.0 + m0
        h1x = jnp.dot(m1x * one_b0, wm1_ref[...], preferred_element_type=jnp.float32)
        h1y = jnp.dot(m1y * one_b0, wm1_ref[...], preferred_element_type=jnp.float32)
        h1z = jnp.dot(m1z * one_b0, wm1_ref[...], preferred_element_type=jnp.float32)
        pr = jnp.sum(jnp.dot(h0, wp0_ref[...], preferred_element_type=jnp.float32) * h0,
                     axis=1, keepdims=True)
        for h1 in (h1x, h1y, h1z):
            pr = pr + jnp.sum(
                jnp.dot(h1, wp1_ref[...], preferred_element_type=jnp.float32) * h1,
                axis=1, keepdims=True)
        bat = bat_ref[...]

        @pl.when(i == 0)
        def _():
            sum_ref[...] = jnp.zeros((8, 1), jnp.float32)
            cnt_ref[...] = jnp.zeros((8, 1), jnp.float32)

        for g in range(8):
            m = bat == g
            sum_ref[g:g + 1, :] += jnp.sum(jnp.where(m, pr, 0.0), axis=0, keepdims=True)
            cnt_ref[g:g + 1, :] += jnp.sum(jnp.where(m, 1.0, 0.0), axis=0, keepdims=True)

    return pl.pallas_call(
        body,
        grid=(_N // _NB_NODE,),
        in_specs=[
            pl.BlockSpec((_NBLK, _NB_NODE, _C), lambda i: (0, i, 0)),
            pl.BlockSpec((_NB_NODE, da), lambda i: (i, 0)),
            pl.BlockSpec((_C, _C), lambda i: (0, 0)),
            pl.BlockSpec((_C, _C), lambda i: (0, 0)),
            pl.BlockSpec((da, _C), lambda i: (0, 0)),
            pl.BlockSpec((_C, _C), lambda i: (0, 0)),
            pl.BlockSpec((_C, _C), lambda i: (0, 0)),
            pl.BlockSpec((_NB_NODE, 1), lambda i: (i, 0)),
        ],
        out_specs=[
            pl.BlockSpec((8, 1), lambda i: (0, 0)),
            pl.BlockSpec((8, 1), lambda i: (0, 0)),
        ],
        out_shape=[
            jax.ShapeDtypeStruct((8, 1), jnp.float32),
            jax.ShapeDtypeStruct((8, 1), jnp.float32),
        ],
    )(bacc, attrs, Wm0, Wm1, Ws, Wp0, Wp1, batch2d)


def _sc_params():
    return pltpu.CompilerParams(needs_layout_passes=False,
                                use_tc_tiling_on_sc=False)


def _sc_mesh():
    return plsc.VectorSubcoreMesh(core_axis_name="c", subcore_axis_name="s")


def _unit_y1_16(px, py, pz, sv, dv):
    dx = plsc.load_gather(px, [dv]) - plsc.load_gather(px, [sv])
    dy = plsc.load_gather(py, [dv]) - plsc.load_gather(py, [sv])
    dz = plsc.load_gather(pz, [dv]) - plsc.load_gather(pz, [sv])
    r2 = dx * dx + dy * dy + dz * dz
    ii = plsc.bitcast(r2, jnp.int32)
    ii = 0x5F3759DF - jnp.right_shift(ii, 1)
    y = plsc.bitcast(ii, jnp.float32)
    for _ in range(3):
        y = y * (1.5 - 0.5 * r2 * y * y)
    r = r2 * y
    rc = _C1 / (r + 1e-9)
    return dx * rc, dy * rc, dz * rc


def _sc_y1(posx, posy, posz, edge_index):
    @functools.partial(
        pl.kernel,
        out_type=jax.ShapeDtypeStruct((3, _E), jnp.float32),
        mesh=_sc_mesh(),
        compiler_params=_sc_params(),
        scratch_types=[
            pltpu.VMEM((_NPAD,), jnp.float32),
            pltpu.VMEM((_NPAD,), jnp.float32),
            pltpu.VMEM((_NPAD,), jnp.float32),
            pltpu.VMEM((2, _K), jnp.int32),
            pltpu.VMEM((3, _K), jnp.float32),
        ],
    )
    def k(posx_h, posy_h, posz_h, ei_h, oy_h,
          px, py, pz, ei_v, y1_v):
        c = lax.axis_index("c")
        s = lax.axis_index("s")
        wid = c * _TILES + s
        pltpu.sync_copy(posx_h, px)
        pltpu.sync_copy(posy_h, py)
        pltpu.sync_copy(posz_h, pz)
        nchunks = (_NCHUNK - wid + 2 * _TILES - 1) // (2 * _TILES)

        def chunk(i, carry):
            e0 = (wid + 2 * _TILES * i) * _K
            pltpu.sync_copy(ei_h.at[:, pl.ds(e0, _K)], ei_v)
            for kk in range(8):
                sl = pl.ds(16 * kk, 16)
                yx, yy, yz = _unit_y1_16(px, py, pz, ei_v[0, sl], ei_v[1, sl])
                y1_v[0, sl] = yx
                y1_v[1, sl] = yy
                y1_v[2, sl] = yz
            pltpu.sync_copy(y1_v, oy_h.at[:, pl.ds(e0, _K)])
            return carry

        lax.fori_loop(0, nchunks, chunk, 0)

    return k(posx, posy, posz, edge_index)


def _sc_edge(y1, edge_index, nf4, w4):
    buf_t = [
        pltpu.VMEM((2, _K), jnp.int32),
        pltpu.VMEM((_K,), jnp.int32),
        pltpu.VMEM((_K, _CB), jnp.float32),
        pltpu.VMEM((_K, 64), jnp.float32),
        pltpu.VMEM((3, _K), jnp.float32),
    ]

    @functools.partial(
        pl.kernel,
        out_type=jax.ShapeDtypeStruct((_NBLK * _NPAD, _C), jnp.float32),
        mesh=_sc_mesh(),
        compiler_params=_sc_params(),
        scratch_types=buf_t + buf_t + [
            pltpu.VMEM((_K, _C), jnp.float32),
            pltpu.VMEM_SHARED((_NPAD, _C), jnp.float32),
            pltpu.SemaphoreType.DMA,
            pltpu.SemaphoreType.DMA,
            pltpu.SemaphoreType.DMA,
            pltpu.SemaphoreType.DMA,
        ],
    )
    def k(y1_h, ei_h, nf_h, w_h, out_h,
          ei_a, idx_a, fs_a, wv_a, y1_a,
          ei_b, idx_b, fs_b, wv_b, y1_b,
          out_v, acc, sem_a, sem_b, semg_a, semg_b):
        c = lax.axis_index("c")
        s = lax.axis_index("s")
        bufs_a = (ei_a, idx_a, fs_a, wv_a, y1_a, sem_a, semg_a)
        bufs_b = (ei_b, idx_b, fs_b, wv_b, y1_b, sem_b, semg_b)

        for p in range(2):
            b = 2 * c + p
            bn = b * _NPAD
            bE = b * _E

            def issue_inputs(i, bufs):
                ei_v, idx2, fs, wv, y1_v, sem, semg = bufs
                e0 = (s + _TILES * i) * _K
                pltpu.async_copy(ei_h.at[:, pl.ds(e0, _K)], ei_v, sem)
                pltpu.async_copy(w_h.at[pl.ds(bE + e0, _K), :], wv, sem)
                pltpu.async_copy(y1_h.at[:, pl.ds(e0, _K)], y1_v, sem)

            def wait_inputs_prep_gather(bufs):
                ei_v, idx2, fs, wv, y1_v, sem, semg = bufs
                pltpu.make_async_copy(ei_h.at[:, pl.ds(0, _K)], ei_v, sem).wait()
                pltpu.make_async_copy(w_h.at[pl.ds(0, _K), :], wv, sem).wait()
                pltpu.make_async_copy(y1_h.at[:, pl.ds(0, _K)], y1_v, sem).wait()
                for kk in range(8):
                    sl = pl.ds(16 * kk, 16)
                    idx2[sl] = ei_v[0, sl] + bn
                pltpu.async_copy(nf_h.at[idx2], fs, semg)

            def process(bufs):
                ei_v, idx2, fs, wv, y1_v, sem, semg = bufs
                pltpu.make_async_copy(nf_h.at[idx2], fs, semg).wait()

                def edge(jj, carry2):
                    for u in range(2):
                        j = 2 * jj + u
                        f0 = fs[j, pl.ds(0, 16)]
                        f1 = fs[j, pl.ds(16, 16)]
                        w00 = wv[j, pl.ds(0, 16)]
                        w01 = wv[j, pl.ds(16, 16)]
                        w10 = wv[j, pl.ds(32, 16)]
                        w11 = wv[j, pl.ds(48, 16)]
                        jj16 = jnp.full((16,), j, jnp.int32)
                        z16 = jnp.zeros((16,), jnp.int32)
                        yx = plsc.load_gather(y1_v, [z16, jj16])
                        yy = plsc.load_gather(y1_v, [z16 + 1, jj16])
                        yz = plsc.load_gather(y1_v, [z16 + 2, jj16])
                        t0 = f0 * w10
                        t1 = f1 * w11
                        out_v[j, pl.ds(0, 16)] = f0 * w00
                        out_v[j, pl.ds(16, 16)] = f1 * w01
                        out_v[j, pl.ds(32, 16)] = t0 * yx
                        out_v[j, pl.ds(48, 16)] = t1 * yx
                        out_v[j, pl.ds(64, 16)] = t0 * yy
                        out_v[j, pl.ds(80, 16)] = t1 * yy
                        out_v[j, pl.ds(96, 16)] = t0 * yz
                        out_v[j, pl.ds(112, 16)] = t1 * yz
                    return carry2

                lax.fori_loop(0, _K // 2, edge, 0)
                pltpu.sync_copy(out_v, acc.at[ei_v.at[1]], add=True)

            def zero_row(r, carry):
                for kk in range(8):
                    out_v[r, pl.ds(16 * kk, 16)] = jnp.zeros((16,), jnp.float32)
                return carry

            lax.fori_loop(0, _K, zero_row, 0)
            for t in range(5):
                row0 = s * _ROWS_PER_TILE + t * _K
                pltpu.sync_copy(out_v.at[pl.ds(0, _K), :],
                                acc.at[pl.ds(row0, _K), :])
            plsc.subcore_barrier()

            nchunks = (_NCHUNK - s + _TILES - 1) // _TILES

            issue_inputs(0, bufs_a)
            wait_inputs_prep_gather(bufs_a)
            issue_inputs(1, bufs_b)

            def body(i, cur, nxt):
                @pl.when(i + 1 < nchunks)
                def _():
                    wait_inputs_prep_gather(nxt)
                process(cur)

                @pl.when(i + 2 < nchunks)
                def _():
                    issue_inputs(i + 2, cur)

            def chunk(i, carry):
                @pl.when(lax.rem(i, 2) == 0)
                def _():
                    body(i, bufs_a, bufs_b)

                @pl.when(lax.rem(i, 2) == 1)
                def _():
                    body(i, bufs_b, bufs_a)

                return carry

            lax.fori_loop(0, nchunks, chunk, 0)
            plsc.subcore_barrier()
            for t in range(5):
                row0 = s * _ROWS_PER_TILE + t * _K
                pltpu.sync_copy(acc.at[pl.ds(row0, _K), :],
                                out_h.at[pl.ds(b * _NPAD + row0, _K), :])

    return k(y1, edge_index, nf4, w4)


def _pad_feats(f):
    return jnp.pad(f, ((0, 0), (0, _NPAD - _N), (0, 0))).reshape(_NBLK * _NPAD, _CB)


def _perms():
    colperm = np.concatenate([
        np.concatenate([np.arange(32 * b, 32 * b + 32),
                        128 + np.arange(32 * b, 32 * b + 32)])
        for b in range(_NBLK)])
    rowperm = np.concatenate([
        np.arange(128),
        np.array([128 + 3 * cc + ax for ax in range(3) for cc in range(128)])])
    return colperm, rowperm


def kernel(x, pos, edge_attr, W1, b1, R1a, R1b, Wm0_1, Wm1_1, Ws1,
           W2, b2, R2a, R2b, Wm0_2, Wm1_2, Ws2, Wp0, Wp1, edge_index, batch):
    colperm, rowperm = _perms()
    edge_index = edge_index.astype(jnp.int32)
    pos_p = jnp.pad(pos, ((0, _NPAD - _N), (0, 0)))
    posx = pos_p[:, 0]
    posy = pos_p[:, 1]
    posz = pos_p[:, 2]

    y1 = _sc_y1(posx, posy, posz, edge_index)

    f1 = _pad_feats(_feats_tc(x, W1, b1))
    r1 = _radial_tc(edge_attr, R1a, R1b[:, colperm]).reshape(_NBLK * _E, 64)
    bacc1 = _sc_edge(y1, edge_index, f1, r1)
    bacc1 = bacc1.reshape(_NBLK, _NPAD, _C)[:, :_N, :]
    u1 = _node_tc(bacc1, x, Wm0_1, Wm1_1, Ws1)

    f2 = _pad_feats(_feats_tc(u1, W2[rowperm], b2))
    r2 = _radial_tc(edge_attr, R2a, R2b[:, colperm]).reshape(_NBLK * _E, 64)
    bacc2 = _sc_edge(y1, edge_index, f2, r2)
    bacc2 = bacc2.reshape(_NBLK, _NPAD, _C)[:, :_N, :]
    sums, counts = _node_final_tc(bacc2, u1,
                                  Wm0_2, Wm1_2, Ws2[rowperm], Wp0, Wp1,
                                  batch.reshape(_N, 1).astype(jnp.int32))
    return sums / jnp.maximum(counts, 1.0)


# speedup vs baseline: 3.7678x; 1.1147x over previous
import functools

import jax
import jax.numpy as jnp
import numpy as np
from jax import lax
from jax.experimental import pallas as pl
from jax.experimental.pallas import tpu as pltpu
from jax.experimental.pallas import tpu_sc as plsc

_N = 10000
_E = 320000
_C = 128
_CB = 32
_NBLK = 4
_K = 128
_NCHUNK = _E // _K
_TILES = 16
_NPAD = 10240
_ROWS_PER_TILE = _NPAD // _TILES
_C1 = float(np.sqrt(3.0 / (4.0 * np.pi)))

_NB_NODE = 2000
_EB = 8000


def _dot_t(a, bt):
    return lax.dot_general(a, bt, (((1,), (1,)), ((), ())),
                           preferred_element_type=jnp.float32)


def _feats_tc(x, W, b):
    da = x.shape[1]
    wt = W.T.reshape(_NBLK, _CB, da)

    def body(x_ref, w_ref, b_ref, o_ref):
        j = pl.program_id(1)
        val = _dot_t(x_ref[...], w_ref[0])
        o_ref[...] = (val + b_ref[pl.ds(j, 1), :])[None]

    return pl.pallas_call(
        body,
        grid=(_N // _NB_NODE, _NBLK),
        in_specs=[
            pl.BlockSpec((_NB_NODE, da), lambda i, j: (i, 0)),
            pl.BlockSpec((1, _CB, da), lambda i, j: (j, 0, 0)),
            pl.BlockSpec((_NBLK, _CB), lambda i, j: (0, 0)),
        ],
        out_specs=pl.BlockSpec((1, _NB_NODE, _CB), lambda i, j: (j, i, 0)),
        out_shape=jax.ShapeDtypeStruct((_NBLK, _N, _CB), jnp.float32),
    )(x, wt, b.reshape(_NBLK, _CB))


def _radial_tc2(ea, R1a, Rb1p, R2a, Rb2p):
    rbt1 = Rb1p.T.reshape(_NBLK, 64, 64)
    rbt2 = Rb2p.T.reshape(_NBLK, 64, 64)

    def body(e_ref, ra1_ref, rb1_ref, ra2_ref, rb2_ref, o1_ref, o2_ref):
        ea_v = e_ref[...]
        h1 = jax.nn.silu(
            jnp.dot(ea_v, ra1_ref[...], preferred_element_type=jnp.float32))
        o1_ref[...] = _dot_t(h1, rb1_ref[0])[None]
        h2 = jax.nn.silu(
            jnp.dot(ea_v, ra2_ref[...], preferred_element_type=jnp.float32))
        o2_ref[...] = _dot_t(h2, rb2_ref[0])[None]

    out = pl.pallas_call(
        body,
        grid=(_E // _EB, _NBLK),
        in_specs=[
            pl.BlockSpec((_EB, 16), lambda i, j: (i, 0)),
            pl.BlockSpec((16, 64), lambda i, j: (0, 0)),
            pl.BlockSpec((1, 64, 64), lambda i, j: (j, 0, 0)),
            pl.BlockSpec((16, 64), lambda i, j: (0, 0)),
            pl.BlockSpec((1, 64, 64), lambda i, j: (j, 0, 0)),
        ],
        out_specs=[
            pl.BlockSpec((1, _EB, 64), lambda i, j: (j, i, 0)),
            pl.BlockSpec((1, _EB, 64), lambda i, j: (j, i, 0)),
        ],
        out_shape=[
            jax.ShapeDtypeStruct((_NBLK, _E, 64), jnp.float32),
            jax.ShapeDtypeStruct((_NBLK, _E, 64), jnp.float32),
        ],
    )(ea, R1a, rbt1, R2a, rbt2)
    return out[0], out[1]


def _mace_node_parts(b_ref):
    bv = b_ref[...]
    m0 = jnp.concatenate([bv[k, :, 0:32] for k in range(_NBLK)], axis=-1)
    m1x = jnp.concatenate([bv[k, :, 32:64] for k in range(_NBLK)], axis=-1)
    m1y = jnp.concatenate([bv[k, :, 64:96] for k in range(_NBLK)], axis=-1)
    m1z = jnp.concatenate([bv[k, :, 96:128] for k in range(_NBLK)], axis=-1)
    return m0, m1x, m1y, m1z


def _node_tc(bacc, attrs, Wm0, Wm1, Ws, W2p, b2):
    da = attrs.shape[1]
    w2t = W2p.T.reshape(_NBLK, _CB, 4 * _C)

    def body(b_ref, a_ref, wm0_ref, wm1_ref, ws_ref, w2_ref, b2_ref,
             o_ref, f_ref):
        m0, m1x, m1y, m1z = _mace_node_parts(b_ref)
        s = m0 + m0 * m0 + m1x * m1x + m1y * m1y + m1z * m1z
        h0 = (jnp.dot(s, wm0_ref[...], preferred_element_type=jnp.float32)
              + jnp.dot(a_ref[...], ws_ref[...], preferred_element_type=jnp.float32))
        one_b0 = 1.0 + m0
        h1x = jnp.dot(m1x * one_b0, wm1_ref[...], preferred_element_type=jnp.float32)
        h1y = jnp.dot(m1y * one_b0, wm1_ref[...], preferred_element_type=jnp.float32)
        h1z = jnp.dot(m1z * one_b0, wm1_ref[...], preferred_element_type=jnp.float32)
        u = jnp.concatenate([h0, h1x, h1y, h1z], axis=-1)
        o_ref[...] = u
        f_ref[...] = jnp.stack(
            [_dot_t(u, w2_ref[k]) + b2_ref[pl.ds(k, 1), :] for k in range(_NBLK)],
            axis=0)

    return pl.pallas_call(
        body,
        grid=(_N // _NB_NODE,),
        in_specs=[
            pl.BlockSpec((_NBLK, _NB_NODE, _C), lambda i: (0, i, 0)),
            pl.BlockSpec((_NB_NODE, da), lambda i: (i, 0)),
            pl.BlockSpec((_C, _C), lambda i: (0, 0)),
            pl.BlockSpec((_C, _C), lambda i: (0, 0)),
            pl.BlockSpec((da, _C), lambda i: (0, 0)),
            pl.BlockSpec((_NBLK, _CB, 4 * _C), lambda i: (0, 0, 0)),
            pl.BlockSpec((_NBLK, _CB), lambda i: (0, 0)),
        ],
        out_specs=[
            pl.BlockSpec((_NB_NODE, 4 * _C), lambda i: (i, 0)),
            pl.BlockSpec((_NBLK, _NB_NODE, _CB), lambda i: (0, i, 0)),
        ],
        out_shape=[
            jax.ShapeDtypeStruct((_N, 4 * _C), jnp.float32),
            jax.ShapeDtypeStruct((_NBLK, _N, _CB), jnp.float32),
        ],
    )(bacc, attrs, Wm0, Wm1, Ws, w2t, b2.reshape(_NBLK, _CB))


def _node_final_tc(bacc, attrs, Wm0, Wm1, Ws, Wp0, Wp1, batch2d):
    da = attrs.shape[1]

    def body(b_ref, a_ref, wm0_ref, wm1_ref, ws_ref, wp0_ref, wp1_ref,
             bat_ref, sum_ref, cnt_ref):
        i = pl.program_id(0)
        m0, m1x, m1y, m1z = _mace_node_parts(b_ref)
        s = m0 + m0 * m0 + m1x * m1x + m1y * m1y + m1z * m1z
        h0 = (jnp.dot(s, wm0_ref[...], preferred_element_type=jnp.float32)
              + jnp.dot(a_ref[...], ws_ref[...], preferred_element_type=jnp.float32))
        one_b0 = 1.0 + m0
        h1x = jnp.dot(m1x * one_b0, wm1_ref[...], preferred_element_type=jnp.float32)
        h1y = jnp.dot(m1y * one_b0, wm1_ref[...], preferred_element_type=jnp.float32)
        h1z = jnp.dot(m1z * one_b0, wm1_ref[...], preferred_element_type=jnp.float32)
        pr = jnp.sum(jnp.dot(h0, wp0_ref[...], preferred_element_type=jnp.float32) * h0,
                     axis=1, keepdims=True)
        for h1 in (h1x, h1y, h1z):
            pr = pr + jnp.sum(
                jnp.dot(h1, wp1_ref[...], preferred_element_type=jnp.float32) * h1,
                axis=1, keepdims=True)
        bat = bat_ref[...]

        @pl.when(i == 0)
        def _():
            sum_ref[...] = jnp.zeros((8, 1), jnp.float32)
            cnt_ref[...] = jnp.zeros((8, 1), jnp.float32)

        for g in range(8):
            m = bat == g
            sum_ref[g:g + 1, :] += jnp.sum(jnp.where(m, pr, 0.0), axis=0, keepdims=True)
            cnt_ref[g:g + 1, :] += jnp.sum(jnp.where(m, 1.0, 0.0), axis=0, keepdims=True)

    return pl.pallas_call(
        body,
        grid=(_N // _NB_NODE,),
        in_specs=[
            pl.BlockSpec((_NBLK, _NB_NODE, _C), lambda i: (0, i, 0)),
            pl.BlockSpec((_NB_NODE, da), lambda i: (i, 0)),
            pl.BlockSpec((_C, _C), lambda i: (0, 0)),
            pl.BlockSpec((_C, _C), lambda i: (0, 0)),
            pl.BlockSpec((da, _C), lambda i: (0, 0)),
            pl.BlockSpec((_C, _C), lambda i: (0, 0)),
            pl.BlockSpec((_C, _C), lambda i: (0, 0)),
            pl.BlockSpec((_NB_NODE, 1), lambda i: (i, 0)),
        ],
        out_specs=[
            pl.BlockSpec((8, 1), lambda i: (0, 0)),
            pl.BlockSpec((8, 1), lambda i: (0, 0)),
        ],
        out_shape=[
            jax.ShapeDtypeStruct((8, 1), jnp.float32),
            jax.ShapeDtypeStruct((8, 1), jnp.float32),
        ],
    )(bacc, attrs, Wm0, Wm1, Ws, Wp0, Wp1, batch2d)


def _sc_params():
    return pltpu.CompilerParams(needs_layout_passes=False,
                                use_tc_tiling_on_sc=False)


def _sc_mesh():
    return plsc.VectorSubcoreMesh(core_axis_name="c", subcore_axis_name="s")


def _unit_y1_16(px, py, pz, sv, dv):
    dx = plsc.load_gather(px, [dv]) - plsc.load_gather(px, [sv])
    dy = plsc.load_gather(py, [dv]) - plsc.load_gather(py, [sv])
    dz = plsc.load_gather(pz, [dv]) - plsc.load_gather(pz, [sv])
    r2 = dx * dx + dy * dy + dz * dz
    ii = plsc.bitcast(r2, jnp.int32)
    ii = 0x5F3759DF - jnp.right_shift(ii, 1)
    y = plsc.bitcast(ii, jnp.float32)
    for _ in range(3):
        y = y * (1.5 - 0.5 * r2 * y * y)
    r = r2 * y
    rc = _C1 / (r + 1e-9)
    return dx * rc, dy * rc, dz * rc


def _sc_y1(posx, posy, posz, edge_index):
    @functools.partial(
        pl.kernel,
        out_type=jax.ShapeDtypeStruct((3, _E), jnp.float32),
        mesh=_sc_mesh(),
        compiler_params=_sc_params(),
        scratch_types=[
            pltpu.VMEM((_NPAD,), jnp.float32),
            pltpu.VMEM((_NPAD,), jnp.float32),
            pltpu.VMEM((_NPAD,), jnp.float32),
            pltpu.VMEM((2, _K), jnp.int32),
            pltpu.VMEM((3, _K), jnp.float32),
        ],
    )
    def k(posx_h, posy_h, posz_h, ei_h, oy_h,
          px, py, pz, ei_v, y1_v):
        c = lax.axis_index("c")
        s = lax.axis_index("s")
        wid = c * _TILES + s
        pltpu.sync_copy(posx_h, px)
        pltpu.sync_copy(posy_h, py)
        pltpu.sync_copy(posz_h, pz)
        nchunks = (_NCHUNK - wid + 2 * _TILES - 1) // (2 * _TILES)

        def chunk(i, carry):
            e0 = (wid + 2 * _TILES * i) * _K
            pltpu.sync_copy(ei_h.at[:, pl.ds(e0, _K)], ei_v)
            for kk in range(8):
                sl = pl.ds(16 * kk, 16)
                yx, yy, yz = _unit_y1_16(px, py, pz, ei_v[0, sl], ei_v[1, sl])
                y1_v[0, sl] = yx
                y1_v[1, sl] = yy
                y1_v[2, sl] = yz
            pltpu.sync_copy(y1_v, oy_h.at[:, pl.ds(e0, _K)])
            return carry

        lax.fori_loop(0, nchunks, chunk, 0)

    return k(posx, posy, posz, edge_index)


def _sc_edge(y1, edge_index, nf4, w4):
    buf_t = [
        pltpu.VMEM((2, _K), jnp.int32),
        pltpu.VMEM((_K,), jnp.int32),
        pltpu.VMEM((_K, _CB), jnp.float32),
        pltpu.VMEM((_K, 64), jnp.float32),
        pltpu.VMEM((3, _K), jnp.float32),
    ]

    @functools.partial(
        pl.kernel,
        out_type=jax.ShapeDtypeStruct((_NBLK * _NPAD, _C), jnp.float32),
        mesh=_sc_mesh(),
        compiler_params=_sc_params(),
        scratch_types=buf_t + buf_t + [
            pltpu.VMEM((_K, _C), jnp.float32),
            pltpu.VMEM((_K,), jnp.int32),
            pltpu.VMEM_SHARED((_NPAD, _C), jnp.float32),
            pltpu.SemaphoreType.DMA,
            pltpu.SemaphoreType.DMA,
            pltpu.SemaphoreType.DMA,
            pltpu.SemaphoreType.DMA,
            pltpu.SemaphoreType.DMA,
        ],
    )
    def k(y1_h, ei_h, nf_h, w_h, out_h,
          ei_a, idx_a, fs_a, wv_a, y1_a,
          ei_b, idx_b, fs_b, wv_b, y1_b,
          out_v, dst_idx, acc, sem_a, sem_b, semg_a, semg_b, sem_s):
        c = lax.axis_index("c")
        s = lax.axis_index("s")
        bufs_a = (ei_a, idx_a, fs_a, wv_a, y1_a, sem_a, semg_a)
        bufs_b = (ei_b, idx_b, fs_b, wv_b, y1_b, sem_b, semg_b)

        for p in range(2):
            b = 2 * c + p
            bn = b * _NPAD
            bE = b * _E

            def issue_inputs(i, bufs):
                ei_v, idx2, fs, wv, y1_v, sem, semg = bufs
                e0 = (s + _TILES * i) * _K
                pltpu.async_copy(ei_h.at[:, pl.ds(e0, _K)], ei_v, sem)
                pltpu.async_copy(w_h.at[pl.ds(bE + e0, _K), :], wv, sem)
                pltpu.async_copy(y1_h.at[:, pl.ds(e0, _K)], y1_v, sem)

            def wait_inputs_prep_gather(bufs):
                ei_v, idx2, fs, wv, y1_v, sem, semg = bufs
                pltpu.make_async_copy(ei_h.at[:, pl.ds(0, _K)], ei_v, sem).wait()
                pltpu.make_async_copy(w_h.at[pl.ds(0, _K), :], wv, sem).wait()
                pltpu.make_async_copy(y1_h.at[:, pl.ds(0, _K)], y1_v, sem).wait()
                for kk in range(8):
                    sl = pl.ds(16 * kk, 16)
                    idx2[sl] = ei_v[0, sl] + bn
                pltpu.async_copy(nf_h.at[idx2], fs, semg)

            def process(bufs, i):
                ei_v, idx2, fs, wv, y1_v, sem, semg = bufs
                pltpu.make_async_copy(nf_h.at[idx2], fs, semg).wait()

                @pl.when(i > 0)
                def _():
                    pltpu.make_async_copy(out_v, acc.at[dst_idx], sem_s).wait()
                for kk in range(8):
                    sl = pl.ds(16 * kk, 16)
                    dst_idx[sl] = ei_v[1, sl]

                def edge(jj, carry2):
                    for u in range(2):
                        j = 2 * jj + u
                        f0 = fs[j, pl.ds(0, 16)]
                        f1 = fs[j, pl.ds(16, 16)]
                        w00 = wv[j, pl.ds(0, 16)]
                        w01 = wv[j, pl.ds(16, 16)]
                        w10 = wv[j, pl.ds(32, 16)]
                        w11 = wv[j, pl.ds(48, 16)]
                        jj16 = jnp.full((16,), j, jnp.int32)
                        z16 = jnp.zeros((16,), jnp.int32)
                        yx = plsc.load_gather(y1_v, [z16, jj16])
                        yy = plsc.load_gather(y1_v, [z16 + 1, jj16])
                        yz = plsc.load_gather(y1_v, [z16 + 2, jj16])
                        t0 = f0 * w10
                        t1 = f1 * w11
                        out_v[j, pl.ds(0, 16)] = f0 * w00
                        out_v[j, pl.ds(16, 16)] = f1 * w01
                        out_v[j, pl.ds(32, 16)] = t0 * yx
                        out_v[j, pl.ds(48, 16)] = t1 * yx
                        out_v[j, pl.ds(64, 16)] = t0 * yy
                        out_v[j, pl.ds(80, 16)] = t1 * yy
                        out_v[j, pl.ds(96, 16)] = t0 * yz
                        out_v[j, pl.ds(112, 16)] = t1 * yz
                    return carry2

                lax.fori_loop(0, _K // 2, edge, 0)
                pltpu.async_copy(out_v, acc.at[dst_idx], sem_s, add=True)

            def zero_row(r, carry):
                for kk in range(8):
                    out_v[r, pl.ds(16 * kk, 16)] = jnp.zeros((16,), jnp.float32)
                return carry

            lax.fori_loop(0, _K, zero_row, 0)
            for t in range(5):
                row0 = s * _ROWS_PER_TILE + t * _K
                pltpu.sync_copy(out_v.at[pl.ds(0, _K), :],
                                acc.at[pl.ds(row0, _K), :])
            plsc.subcore_barrier()

            nchunks = (_NCHUNK - s + _TILES - 1) // _TILES

            issue_inputs(0, bufs_a)
            wait_inputs_prep_gather(bufs_a)
            issue_inputs(1, bufs_b)

            def body(i, cur, nxt):
                @pl.when(i + 1 < nchunks)
                def _():
                    wait_inputs_prep_gather(nxt)
                process(cur, i)

                @pl.when(i + 2 < nchunks)
                def _():
                    issue_inputs(i + 2, cur)

            def chunk(i, carry):
                @pl.when(lax.rem(i, 2) == 0)
                def _():
                    body(i, bufs_a, bufs_b)

                @pl.when(lax.rem(i, 2) == 1)
                def _():
                    body(i, bufs_b, bufs_a)

                return carry

            lax.fori_loop(0, nchunks, chunk, 0)
            pltpu.make_async_copy(out_v, acc.at[dst_idx], sem_s).wait()
            plsc.subcore_barrier()
            for t in range(5):
                row0 = s * _ROWS_PER_TILE + t * _K
                pltpu.sync_copy(acc.at[pl.ds(row0, _K), :],
                                out_h.at[pl.ds(b * _NPAD + row0, _K), :])

    return k(y1, edge_index, nf4, w4)


def _pad_feats(f):
    return jnp.pad(f, ((0, 0), (0, _NPAD - _N), (0, 0))).reshape(_NBLK * _NPAD, _CB)


def _perms():
    colperm = np.concatenate([
        np.concatenate([np.arange(32 * b, 32 * b + 32),
                        128 + np.arange(32 * b, 32 * b + 32)])
        for b in range(_NBLK)])
    rowperm = np.concatenate([
        np.arange(128),
        np.array([128 + 3 * cc + ax for ax in range(3) for cc in range(128)])])
    return colperm, rowperm


def kernel(x, pos, edge_attr, W1, b1, R1a, R1b, Wm0_1, Wm1_1, Ws1,
           W2, b2, R2a, R2b, Wm0_2, Wm1_2, Ws2, Wp0, Wp1, edge_index, batch):
    colperm, rowperm = _perms()
    edge_index = edge_index.astype(jnp.int32)
    pos_p = jnp.pad(pos, ((0, _NPAD - _N), (0, 0)))
    posx = pos_p[:, 0]
    posy = pos_p[:, 1]
    posz = pos_p[:, 2]

    y1 = _sc_y1(posx, posy, posz, edge_index)

    r1, r2 = _radial_tc2(edge_attr, R1a, R1b[:, colperm],
                         R2a, R2b[:, colperm])
    r1 = r1.reshape(_NBLK * _E, 64)
    r2 = r2.reshape(_NBLK * _E, 64)

    f1 = _pad_feats(_feats_tc(x, W1, b1))
    bacc1 = _sc_edge(y1, edge_index, f1, r1)
    bacc1 = bacc1.reshape(_NBLK, _NPAD, _C)[:, :_N, :]
    u1, f2 = _node_tc(bacc1, x, Wm0_1, Wm1_1, Ws1, W2[rowperm], b2)
    f2 = _pad_feats(f2)

    bacc2 = _sc_edge(y1, edge_index, f2, r2)
    bacc2 = bacc2.reshape(_NBLK, _NPAD, _C)[:, :_N, :]
    sums, counts = _node_final_tc(bacc2, u1,
                                  Wm0_2, Wm1_2, Ws2[rowperm], Wp0, Wp1,
                                  batch.reshape(_N, 1).astype(jnp.int32))
    return sums / jnp.maximum(counts, 1.0)


# speedup vs baseline: 3.8442x; 1.0203x over previous
import functools

import jax
import jax.numpy as jnp
import numpy as np
from jax import lax
from jax.experimental import pallas as pl
from jax.experimental.pallas import tpu as pltpu
from jax.experimental.pallas import tpu_sc as plsc

_N = 10000
_E = 320000
_C = 128
_CB = 32
_NBLK = 4
_K = 128
_NCHUNK = _E // _K
_TILES = 16
_NPAD = 10240
_ROWS_PER_TILE = _NPAD // _TILES
_C1 = float(np.sqrt(3.0 / (4.0 * np.pi)))

_NB_NODE = 2000
_EB = 8000


def _dot_t(a, bt):
    return lax.dot_general(a, bt, (((1,), (1,)), ((), ())),
                           preferred_element_type=jnp.float32)


def _feats_tc(x, W, b):
    da = x.shape[1]
    wt = W.T.reshape(_NBLK, _CB, da)

    def body(x_ref, w_ref, b_ref, o_ref):
        j = pl.program_id(1)
        val = _dot_t(x_ref[...], w_ref[0])
        o_ref[...] = (val + b_ref[pl.ds(j, 1), :])[None]

    return pl.pallas_call(
        body,
        grid=(_N // _NB_NODE, _NBLK),
        in_specs=[
            pl.BlockSpec((_NB_NODE, da), lambda i, j: (i, 0)),
            pl.BlockSpec((1, _CB, da), lambda i, j: (j, 0, 0)),
            pl.BlockSpec((_NBLK, _CB), lambda i, j: (0, 0)),
        ],
        out_specs=pl.BlockSpec((1, _NB_NODE, _CB), lambda i, j: (j, i, 0)),
        out_shape=jax.ShapeDtypeStruct((_NBLK, _NPAD, _CB), jnp.float32),
    )(x, wt, b.reshape(_NBLK, _CB))


def _radial_tc2(ea, R1a, Rb1p, R2a, Rb2p):
    rbt1 = Rb1p.T.reshape(_NBLK, 64, 64)
    rbt2 = Rb2p.T.reshape(_NBLK, 64, 64)

    def body(e_ref, ra1_ref, rb1_ref, ra2_ref, rb2_ref, o1_ref, o2_ref):
        ea_v = e_ref[...]
        h1 = jax.nn.silu(
            jnp.dot(ea_v, ra1_ref[...], preferred_element_type=jnp.float32))
        o1_ref[...] = _dot_t(h1, rb1_ref[0])[None]
        h2 = jax.nn.silu(
            jnp.dot(ea_v, ra2_ref[...], preferred_element_type=jnp.float32))
        o2_ref[...] = _dot_t(h2, rb2_ref[0])[None]

    out = pl.pallas_call(
        body,
        grid=(_E // _EB, _NBLK),
        in_specs=[
            pl.BlockSpec((_EB, 16), lambda i, j: (i, 0)),
            pl.BlockSpec((16, 64), lambda i, j: (0, 0)),
            pl.BlockSpec((1, 64, 64), lambda i, j: (j, 0, 0)),
            pl.BlockSpec((16, 64), lambda i, j: (0, 0)),
            pl.BlockSpec((1, 64, 64), lambda i, j: (j, 0, 0)),
        ],
        out_specs=[
            pl.BlockSpec((1, _EB, 64), lambda i, j: (j, i, 0)),
            pl.BlockSpec((1, _EB, 64), lambda i, j: (j, i, 0)),
        ],
        out_shape=[
            jax.ShapeDtypeStruct((_NBLK, _E, 64), jnp.float32),
            jax.ShapeDtypeStruct((_NBLK, _E, 64), jnp.float32),
        ],
    )(ea, R1a, rbt1, R2a, rbt2)
    return out[0], out[1]


def _mace_node_parts(b_ref):
    bv = b_ref[...]
    m0 = jnp.concatenate([bv[k, :, 0:32] for k in range(_NBLK)], axis=-1)
    m1x = jnp.concatenate([bv[k, :, 32:64] for k in range(_NBLK)], axis=-1)
    m1y = jnp.concatenate([bv[k, :, 64:96] for k in range(_NBLK)], axis=-1)
    m1z = jnp.concatenate([bv[k, :, 96:128] for k in range(_NBLK)], axis=-1)
    return m0, m1x, m1y, m1z


def _node_tc(bacc, attrs, Wm0, Wm1, Ws, W2p, b2):
    da = attrs.shape[1]
    w2t = W2p.T.reshape(_NBLK, _CB, 4 * _C)

    def body(b_ref, a_ref, wm0_ref, wm1_ref, ws_ref, w2_ref, b2_ref,
             o_ref, f_ref):
        m0, m1x, m1y, m1z = _mace_node_parts(b_ref)
        s = m0 + m0 * m0 + m1x * m1x + m1y * m1y + m1z * m1z
        h0 = (jnp.dot(s, wm0_ref[...], preferred_element_type=jnp.float32)
              + jnp.dot(a_ref[...], ws_ref[...], preferred_element_type=jnp.float32))
        one_b0 = 1.0 + m0
        h1x = jnp.dot(m1x * one_b0, wm1_ref[...], preferred_element_type=jnp.float32)
        h1y = jnp.dot(m1y * one_b0, wm1_ref[...], preferred_element_type=jnp.float32)
        h1z = jnp.dot(m1z * one_b0, wm1_ref[...], preferred_element_type=jnp.float32)
        u = jnp.concatenate([h0, h1x, h1y, h1z], axis=-1)
        o_ref[...] = u
        f_ref[...] = jnp.stack(
            [_dot_t(u, w2_ref[k]) + b2_ref[pl.ds(k, 1), :] for k in range(_NBLK)],
            axis=0)

    return pl.pallas_call(
        body,
        grid=(_N // _NB_NODE,),
        in_specs=[
            pl.BlockSpec((_NBLK, _NB_NODE, _C), lambda i: (0, i, 0)),
            pl.BlockSpec((_NB_NODE, da), lambda i: (i, 0)),
            pl.BlockSpec((_C, _C), lambda i: (0, 0)),
            pl.BlockSpec((_C, _C), lambda i: (0, 0)),
            pl.BlockSpec((da, _C), lambda i: (0, 0)),
            pl.BlockSpec((_NBLK, _CB, 4 * _C), lambda i: (0, 0, 0)),
            pl.BlockSpec((_NBLK, _CB), lambda i: (0, 0)),
        ],
        out_specs=[
            pl.BlockSpec((_NB_NODE, 4 * _C), lambda i: (i, 0)),
            pl.BlockSpec((_NBLK, _NB_NODE, _CB), lambda i: (0, i, 0)),
        ],
        out_shape=[
            jax.ShapeDtypeStruct((_N, 4 * _C), jnp.float32),
            jax.ShapeDtypeStruct((_NBLK, _NPAD, _CB), jnp.float32),
        ],
    )(bacc, attrs, Wm0, Wm1, Ws, w2t, b2.reshape(_NBLK, _CB))


def _node_final_tc(bacc, attrs, Wm0, Wm1, Ws, Wp0, Wp1, batch2d):
    da = attrs.shape[1]

    def body(b_ref, a_ref, wm0_ref, wm1_ref, ws_ref, wp0_ref, wp1_ref,
             bat_ref, sum_ref, cnt_ref):
        i = pl.program_id(0)
        m0, m1x, m1y, m1z = _mace_node_parts(b_ref)
        s = m0 + m0 * m0 + m1x * m1x + m1y * m1y + m1z * m1z
        h0 = (jnp.dot(s, wm0_ref[...], preferred_element_type=jnp.float32)
              + jnp.dot(a_ref[...], ws_ref[...], preferred_element_type=jnp.float32))
        one_b0 = 1.0 + m0
        h1x = jnp.dot(m1x * one_b0, wm1_ref[...], preferred_element_type=jnp.float32)
        h1y = jnp.dot(m1y * one_b0, wm1_ref[...], preferred_element_type=jnp.float32)
        h1z = jnp.dot(m1z * one_b0, wm1_ref[...], preferred_element_type=jnp.float32)
        pr = jnp.sum(jnp.dot(h0, wp0_ref[...], preferred_element_type=jnp.float32) * h0,
                     axis=1, keepdims=True)
        for h1 in (h1x, h1y, h1z):
            pr = pr + jnp.sum(
                jnp.dot(h1, wp1_ref[...], preferred_element_type=jnp.float32) * h1,
                axis=1, keepdims=True)
        bat = bat_ref[...]

        @pl.when(i == 0)
        def _():
            sum_ref[...] = jnp.zeros((8, 1), jnp.float32)
            cnt_ref[...] = jnp.zeros((8, 1), jnp.float32)

        for g in range(8):
            m = bat == g
            sum_ref[g:g + 1, :] += jnp.sum(jnp.where(m, pr, 0.0), axis=0, keepdims=True)
            cnt_ref[g:g + 1, :] += jnp.sum(jnp.where(m, 1.0, 0.0), axis=0, keepdims=True)

    return pl.pallas_call(
        body,
        grid=(_N // _NB_NODE,),
        in_specs=[
            pl.BlockSpec((_NBLK, _NB_NODE, _C), lambda i: (0, i, 0)),
            pl.BlockSpec((_NB_NODE, da), lambda i: (i, 0)),
            pl.BlockSpec((_C, _C), lambda i: (0, 0)),
            pl.BlockSpec((_C, _C), lambda i: (0, 0)),
            pl.BlockSpec((da, _C), lambda i: (0, 0)),
            pl.BlockSpec((_C, _C), lambda i: (0, 0)),
            pl.BlockSpec((_C, _C), lambda i: (0, 0)),
            pl.BlockSpec((_NB_NODE, 1), lambda i: (i, 0)),
        ],
        out_specs=[
            pl.BlockSpec((8, 1), lambda i: (0, 0)),
            pl.BlockSpec((8, 1), lambda i: (0, 0)),
        ],
        out_shape=[
            jax.ShapeDtypeStruct((8, 1), jnp.float32),
            jax.ShapeDtypeStruct((8, 1), jnp.float32),
        ],
    )(bacc, attrs, Wm0, Wm1, Ws, Wp0, Wp1, batch2d)


def _sc_params():
    return pltpu.CompilerParams(needs_layout_passes=False,
                                use_tc_tiling_on_sc=False)


def _sc_mesh():
    return plsc.VectorSubcoreMesh(core_axis_name="c", subcore_axis_name="s")


def _unit_y1_16(px, py, pz, sv, dv):
    dx = plsc.load_gather(px, [dv]) - plsc.load_gather(px, [sv])
    dy = plsc.load_gather(py, [dv]) - plsc.load_gather(py, [sv])
    dz = plsc.load_gather(pz, [dv]) - plsc.load_gather(pz, [sv])
    r2 = dx * dx + dy * dy + dz * dz
    ii = plsc.bitcast(r2, jnp.int32)
    ii = 0x5F3759DF - jnp.right_shift(ii, 1)
    y = plsc.bitcast(ii, jnp.float32)
    for _ in range(3):
        y = y * (1.5 - 0.5 * r2 * y * y)
    r = r2 * y
    rc = _C1 / (r + 1e-9)
    return dx * rc, dy * rc, dz * rc


def _sc_y1(posx, posy, posz, edge_index):
    @functools.partial(
        pl.kernel,
        out_type=jax.ShapeDtypeStruct((3, _E), jnp.float32),
        mesh=_sc_mesh(),
        compiler_params=_sc_params(),
        scratch_types=[
            pltpu.VMEM((_NPAD,), jnp.float32),
            pltpu.VMEM((_NPAD,), jnp.float32),
            pltpu.VMEM((_NPAD,), jnp.float32),
            pltpu.VMEM((2, _K), jnp.int32),
            pltpu.VMEM((3, _K), jnp.float32),
        ],
    )
    def k(posx_h, posy_h, posz_h, ei_h, oy_h,
          px, py, pz, ei_v, y1_v):
        c = lax.axis_index("c")
        s = lax.axis_index("s")
        wid = c * _TILES + s
        pltpu.sync_copy(posx_h, px)
        pltpu.sync_copy(posy_h, py)
        pltpu.sync_copy(posz_h, pz)
        nchunks = (_NCHUNK - wid + 2 * _TILES - 1) // (2 * _TILES)

        def chunk(i, carry):
            e0 = (wid + 2 * _TILES * i) * _K
            pltpu.sync_copy(ei_h.at[:, pl.ds(e0, _K)], ei_v)
            for kk in range(8):
                sl = pl.ds(16 * kk, 16)
                yx, yy, yz = _unit_y1_16(px, py, pz, ei_v[0, sl], ei_v[1, sl])
                y1_v[0, sl] = yx
                y1_v[1, sl] = yy
                y1_v[2, sl] = yz
            pltpu.sync_copy(y1_v, oy_h.at[:, pl.ds(e0, _K)])
            return carry

        lax.fori_loop(0, nchunks, chunk, 0)

    return k(posx, posy, posz, edge_index)


def _sc_edge(y1, edge_index, nf4, w4):
    buf_t = [
        pltpu.VMEM((2, _K), jnp.int32),
        pltpu.VMEM((_K,), jnp.int32),
        pltpu.VMEM((_K, _CB), jnp.float32),
        pltpu.VMEM((_K, 64), jnp.float32),
        pltpu.VMEM((3, _K), jnp.float32),
    ]

    @functools.partial(
        pl.kernel,
        out_type=jax.ShapeDtypeStruct((_NBLK * _NPAD, _C), jnp.float32),
        mesh=_sc_mesh(),
        compiler_params=_sc_params(),
        scratch_types=buf_t + buf_t + [
            pltpu.VMEM((_K, _C), jnp.float32),
            pltpu.VMEM((_K,), jnp.int32),
            pltpu.VMEM_SHARED((_NPAD, _C), jnp.float32),
            pltpu.SemaphoreType.DMA,
            pltpu.SemaphoreType.DMA,
            pltpu.SemaphoreType.DMA,
            pltpu.SemaphoreType.DMA,
            pltpu.SemaphoreType.DMA,
        ],
    )
    def k(y1_h, ei_h, nf_h, w_h, out_h,
          ei_a, idx_a, fs_a, wv_a, y1_a,
          ei_b, idx_b, fs_b, wv_b, y1_b,
          out_v, dst_idx, acc, sem_a, sem_b, semg_a, semg_b, sem_s):
        c = lax.axis_index("c")
        s = lax.axis_index("s")
        bufs_a = (ei_a, idx_a, fs_a, wv_a, y1_a, sem_a, semg_a)
        bufs_b = (ei_b, idx_b, fs_b, wv_b, y1_b, sem_b, semg_b)

        for p in range(2):
            b = 2 * c + p
            bn = b * _NPAD
            bE = b * _E

            def issue_inputs(i, bufs):
                ei_v, idx2, fs, wv, y1_v, sem, semg = bufs
                e0 = (s + _TILES * i) * _K
                pltpu.async_copy(ei_h.at[:, pl.ds(e0, _K)], ei_v, sem)
                pltpu.async_copy(w_h.at[pl.ds(bE + e0, _K), :], wv, sem)
                pltpu.async_copy(y1_h.at[:, pl.ds(e0, _K)], y1_v, sem)

            def wait_inputs_prep_gather(bufs):
                ei_v, idx2, fs, wv, y1_v, sem, semg = bufs
                pltpu.make_async_copy(ei_h.at[:, pl.ds(0, _K)], ei_v, sem).wait()
                pltpu.make_async_copy(w_h.at[pl.ds(0, _K), :], wv, sem).wait()
                pltpu.make_async_copy(y1_h.at[:, pl.ds(0, _K)], y1_v, sem).wait()
                for kk in range(8):
                    sl = pl.ds(16 * kk, 16)
                    idx2[sl] = ei_v[0, sl] + bn
                pltpu.async_copy(nf_h.at[idx2], fs, semg)

            def process(bufs, i):
                ei_v, idx2, fs, wv, y1_v, sem, semg = bufs
                pltpu.make_async_copy(nf_h.at[idx2], fs, semg).wait()

                @pl.when(i > 0)
                def _():
                    pltpu.make_async_copy(out_v, acc.at[dst_idx], sem_s).wait()
                for kk in range(8):
                    sl = pl.ds(16 * kk, 16)
                    dst_idx[sl] = ei_v[1, sl]

                def edge(jj, carry2):
                    for u in range(2):
                        j = 2 * jj + u
                        f0 = fs[j, pl.ds(0, 16)]
                        f1 = fs[j, pl.ds(16, 16)]
                        w00 = wv[j, pl.ds(0, 16)]
                        w01 = wv[j, pl.ds(16, 16)]
                        w10 = wv[j, pl.ds(32, 16)]
                        w11 = wv[j, pl.ds(48, 16)]
                        jj16 = jnp.full((16,), j, jnp.int32)
                        z16 = jnp.zeros((16,), jnp.int32)
                        yx = plsc.load_gather(y1_v, [z16, jj16])
                        yy = plsc.load_gather(y1_v, [z16 + 1, jj16])
                        yz = plsc.load_gather(y1_v, [z16 + 2, jj16])
                        t0 = f0 * w10
                        t1 = f1 * w11
                        out_v[j, pl.ds(0, 16)] = f0 * w00
                        out_v[j, pl.ds(16, 16)] = f1 * w01
                        out_v[j, pl.ds(32, 16)] = t0 * yx
                        out_v[j, pl.ds(48, 16)] = t1 * yx
                        out_v[j, pl.ds(64, 16)] = t0 * yy
                        out_v[j, pl.ds(80, 16)] = t1 * yy
                        out_v[j, pl.ds(96, 16)] = t0 * yz
                        out_v[j, pl.ds(112, 16)] = t1 * yz
                    return carry2

                lax.fori_loop(0, _K // 2, edge, 0)
                pltpu.async_copy(out_v, acc.at[dst_idx], sem_s, add=True)

            def zero_row(r, carry):
                for kk in range(8):
                    out_v[r, pl.ds(16 * kk, 16)] = jnp.zeros((16,), jnp.float32)
                return carry

            lax.fori_loop(0, _K, zero_row, 0)
            for t in range(5):
                row0 = s * _ROWS_PER_TILE + t * _K
                pltpu.sync_copy(out_v.at[pl.ds(0, _K), :],
                                acc.at[pl.ds(row0, _K), :])
            plsc.subcore_barrier()

            nchunks = (_NCHUNK - s + _TILES - 1) // _TILES

            issue_inputs(0, bufs_a)
            wait_inputs_prep_gather(bufs_a)
            issue_inputs(1, bufs_b)

            def body(i, cur, nxt):
                @pl.when(i + 1 < nchunks)
                def _():
                    wait_inputs_prep_gather(nxt)
                process(cur, i)

                @pl.when(i + 2 < nchunks)
                def _():
                    issue_inputs(i + 2, cur)

            def chunk(i, carry):
                @pl.when(lax.rem(i, 2) == 0)
                def _():
                    body(i, bufs_a, bufs_b)

                @pl.when(lax.rem(i, 2) == 1)
                def _():
                    body(i, bufs_b, bufs_a)

                return carry

            lax.fori_loop(0, nchunks, chunk, 0)
            pltpu.make_async_copy(out_v, acc.at[dst_idx], sem_s).wait()
            plsc.subcore_barrier()
            for t in range(5):
                row0 = s * _ROWS_PER_TILE + t * _K
                pltpu.sync_copy(acc.at[pl.ds(row0, _K), :],
                                out_h.at[pl.ds(b * _NPAD + row0, _K), :])

    return k(y1, edge_index, nf4, w4)


def _perms():
    colperm = np.concatenate([
        np.concatenate([np.arange(32 * b, 32 * b + 32),
                        128 + np.arange(32 * b, 32 * b + 32)])
        for b in range(_NBLK)])
    rowperm = np.concatenate([
        np.arange(128),
        np.array([128 + 3 * cc + ax for ax in range(3) for cc in range(128)])])
    return colperm, rowperm


def kernel(x, pos, edge_attr, W1, b1, R1a, R1b, Wm0_1, Wm1_1, Ws1,
           W2, b2, R2a, R2b, Wm0_2, Wm1_2, Ws2, Wp0, Wp1, edge_index, batch):
    colperm, rowperm = _perms()
    edge_index = edge_index.astype(jnp.int32)
    pos_p = jnp.pad(pos, ((0, _NPAD - _N), (0, 0)))
    posx = pos_p[:, 0]
    posy = pos_p[:, 1]
    posz = pos_p[:, 2]

    y1 = _sc_y1(posx, posy, posz, edge_index)

    r1, r2 = _radial_tc2(edge_attr, R1a, R1b[:, colperm],
                         R2a, R2b[:, colperm])
    r1 = r1.reshape(_NBLK * _E, 64)
    r2 = r2.reshape(_NBLK * _E, 64)

    f1 = _feats_tc(x, W1, b1).reshape(_NBLK * _NPAD, _CB)
    bacc1 = _sc_edge(y1, edge_index, f1, r1)
    u1, f2 = _node_tc(bacc1.reshape(_NBLK, _NPAD, _C), x,
                      Wm0_1, Wm1_1, Ws1, W2[rowperm], b2)

    bacc2 = _sc_edge(y1, edge_index, f2.reshape(_NBLK * _NPAD, _CB), r2)
    sums, counts = _node_final_tc(bacc2.reshape(_NBLK, _NPAD, _C), u1,
                                  Wm0_2, Wm1_2, Ws2[rowperm], Wp0, Wp1,
                                  batch.reshape(_N, 1).astype(jnp.int32))
    return sums / jnp.maximum(counts, 1.0)


# speedup vs baseline: 3.8947x; 1.0131x over previous
import functools

import jax
import jax.numpy as jnp
import numpy as np
from jax import lax
from jax.experimental import pallas as pl
from jax.experimental.pallas import tpu as pltpu
from jax.experimental.pallas import tpu_sc as plsc

_N = 10000
_E = 320000
_C = 128
_CB = 32
_NBLK = 4
_K = 128
_NCHUNK = _E // _K
_TILES = 16
_NPAD = 10240
_ROWS_PER_TILE = _NPAD // _TILES
_C1 = float(np.sqrt(3.0 / (4.0 * np.pi)))

_NB_NODE = 2000
_EB = 8000


def _dot_t(a, bt):
    return lax.dot_general(a, bt, (((1,), (1,)), ((), ())),
                           preferred_element_type=jnp.float32)


def _feats_tc(x, W, b):
    da = x.shape[1]
    wt = W.T.reshape(_NBLK, _CB, da)

    def body(x_ref, w_ref, b_ref, o_ref):
        j = pl.program_id(1)
        val = _dot_t(x_ref[...], w_ref[0])
        o_ref[...] = (val + b_ref[pl.ds(j, 1), :])[None]

    return pl.pallas_call(
        body,
        grid=(_N // _NB_NODE, _NBLK),
        in_specs=[
            pl.BlockSpec((_NB_NODE, da), lambda i, j: (i, 0)),
            pl.BlockSpec((1, _CB, da), lambda i, j: (j, 0, 0)),
            pl.BlockSpec((_NBLK, _CB), lambda i, j: (0, 0)),
        ],
        out_specs=pl.BlockSpec((1, _NB_NODE, _CB), lambda i, j: (j, i, 0)),
        out_shape=jax.ShapeDtypeStruct((_NBLK, _NPAD, _CB), jnp.float32),
    )(x, wt, b.reshape(_NBLK, _CB))


def _radial_tc2(ea, R1a, Rb1p, R2a, Rb2p):
    rbt1 = Rb1p.T.reshape(_NBLK, 64, 64)
    rbt2 = Rb2p.T.reshape(_NBLK, 64, 64)

    def body(e_ref, ra1_ref, rb1_ref, ra2_ref, rb2_ref, o1_ref, o2_ref):
        ea_v = e_ref[...]
        h1 = jax.nn.silu(
            jnp.dot(ea_v, ra1_ref[...], preferred_element_type=jnp.float32))
        o1_ref[...] = _dot_t(h1, rb1_ref[0])[None]
        h2 = jax.nn.silu(
            jnp.dot(ea_v, ra2_ref[...], preferred_element_type=jnp.float32))
        o2_ref[...] = _dot_t(h2, rb2_ref[0])[None]

    out = pl.pallas_call(
        body,
        grid=(_E // _EB, _NBLK),
        in_specs=[
            pl.BlockSpec((_EB, 16), lambda i, j: (i, 0)),
            pl.BlockSpec((16, 64), lambda i, j: (0, 0)),
            pl.BlockSpec((1, 64, 64), lambda i, j: (j, 0, 0)),
            pl.BlockSpec((16, 64), lambda i, j: (0, 0)),
            pl.BlockSpec((1, 64, 64), lambda i, j: (j, 0, 0)),
        ],
        out_specs=[
            pl.BlockSpec((1, _EB, 64), lambda i, j: (j, i, 0)),
            pl.BlockSpec((1, _EB, 64), lambda i, j: (j, i, 0)),
        ],
        out_shape=[
            jax.ShapeDtypeStruct((_NBLK, _E, 64), jnp.float32),
            jax.ShapeDtypeStruct((_NBLK, _E, 64), jnp.float32),
        ],
    )(ea, R1a, rbt1, R2a, rbt2)
    return out[0], out[1]


def _mace_node_parts(b_ref):
    bv = b_ref[...]
    m0 = jnp.concatenate([bv[k, :, 0:32] for k in range(_NBLK)], axis=-1)
    m1x = jnp.concatenate([bv[k, :, 32:64] for k in range(_NBLK)], axis=-1)
    m1y = jnp.concatenate([bv[k, :, 64:96] for k in range(_NBLK)], axis=-1)
    m1z = jnp.concatenate([bv[k, :, 96:128] for k in range(_NBLK)], axis=-1)
    return m0, m1x, m1y, m1z


def _node_tc(bacc, attrs, Wm0, Wm1, Ws, W2p, b2):
    da = attrs.shape[1]
    w2t = W2p.T.reshape(_NBLK, _CB, 4 * _C)

    def body(b_ref, a_ref, wm0_ref, wm1_ref, ws_ref, w2_ref, b2_ref,
             o_ref, f_ref):
        m0, m1x, m1y, m1z = _mace_node_parts(b_ref)
        s = m0 + m0 * m0 + m1x * m1x + m1y * m1y + m1z * m1z
        h0 = (jnp.dot(s, wm0_ref[...], preferred_element_type=jnp.float32)
              + jnp.dot(a_ref[...], ws_ref[...], preferred_element_type=jnp.float32))
        one_b0 = 1.0 + m0
        h1x = jnp.dot(m1x * one_b0, wm1_ref[...], preferred_element_type=jnp.float32)
        h1y = jnp.dot(m1y * one_b0, wm1_ref[...], preferred_element_type=jnp.float32)
        h1z = jnp.dot(m1z * one_b0, wm1_ref[...], preferred_element_type=jnp.float32)
        u = jnp.concatenate([h0, h1x, h1y, h1z], axis=-1)
        o_ref[...] = u
        f_ref[...] = jnp.stack(
            [_dot_t(u, w2_ref[k]) + b2_ref[pl.ds(k, 1), :] for k in range(_NBLK)],
            axis=0)

    return pl.pallas_call(
        body,
        grid=(_N // _NB_NODE,),
        in_specs=[
            pl.BlockSpec((_NBLK, _NB_NODE, _C), lambda i: (0, i, 0)),
            pl.BlockSpec((_NB_NODE, da), lambda i: (i, 0)),
            pl.BlockSpec((_C, _C), lambda i: (0, 0)),
            pl.BlockSpec((_C, _C), lambda i: (0, 0)),
            pl.BlockSpec((da, _C), lambda i: (0, 0)),
            pl.BlockSpec((_NBLK, _CB, 4 * _C), lambda i: (0, 0, 0)),
            pl.BlockSpec((_NBLK, _CB), lambda i: (0, 0)),
        ],
        out_specs=[
            pl.BlockSpec((_NB_NODE, 4 * _C), lambda i: (i, 0)),
            pl.BlockSpec((_NBLK, _NB_NODE, _CB), lambda i: (0, i, 0)),
        ],
        out_shape=[
            jax.ShapeDtypeStruct((_N, 4 * _C), jnp.float32),
            jax.ShapeDtypeStruct((_NBLK, _NPAD, _CB), jnp.float32),
        ],
    )(bacc, attrs, Wm0, Wm1, Ws, w2t, b2.reshape(_NBLK, _CB))


def _node_final_tc(bacc, attrs, Wm0, Wm1, Ws, Wp0, Wp1, batch2d):
    da = attrs.shape[1]

    def body(b_ref, a_ref, wm0_ref, wm1_ref, ws_ref, wp0_ref, wp1_ref,
             bat_ref, sum_ref, cnt_ref):
        i = pl.program_id(0)
        m0, m1x, m1y, m1z = _mace_node_parts(b_ref)
        s = m0 + m0 * m0 + m1x * m1x + m1y * m1y + m1z * m1z
        h0 = (jnp.dot(s, wm0_ref[...], preferred_element_type=jnp.float32)
              + jnp.dot(a_ref[...], ws_ref[...], preferred_element_type=jnp.float32))
        one_b0 = 1.0 + m0
        h1x = jnp.dot(m1x * one_b0, wm1_ref[...], preferred_element_type=jnp.float32)
        h1y = jnp.dot(m1y * one_b0, wm1_ref[...], preferred_element_type=jnp.float32)
        h1z = jnp.dot(m1z * one_b0, wm1_ref[...], preferred_element_type=jnp.float32)
        pr = jnp.sum(jnp.dot(h0, wp0_ref[...], preferred_element_type=jnp.float32) * h0,
                     axis=1, keepdims=True)
        for h1 in (h1x, h1y, h1z):
            pr = pr + jnp.sum(
                jnp.dot(h1, wp1_ref[...], preferred_element_type=jnp.float32) * h1,
                axis=1, keepdims=True)
        bat = bat_ref[...]

        @pl.when(i == 0)
        def _():
            sum_ref[...] = jnp.zeros((8, 1), jnp.float32)
            cnt_ref[...] = jnp.zeros((8, 1), jnp.float32)

        for g in range(8):
            m = bat == g
            sum_ref[g:g + 1, :] += jnp.sum(jnp.where(m, pr, 0.0), axis=0, keepdims=True)
            cnt_ref[g:g + 1, :] += jnp.sum(jnp.where(m, 1.0, 0.0), axis=0, keepdims=True)

    return pl.pallas_call(
        body,
        grid=(_N // _NB_NODE,),
        in_specs=[
            pl.BlockSpec((_NBLK, _NB_NODE, _C), lambda i: (0, i, 0)),
            pl.BlockSpec((_NB_NODE, da), lambda i: (i, 0)),
            pl.BlockSpec((_C, _C), lambda i: (0, 0)),
            pl.BlockSpec((_C, _C), lambda i: (0, 0)),
            pl.BlockSpec((da, _C), lambda i: (0, 0)),
            pl.BlockSpec((_C, _C), lambda i: (0, 0)),
            pl.BlockSpec((_C, _C), lambda i: (0, 0)),
            pl.BlockSpec((_NB_NODE, 1), lambda i: (i, 0)),
        ],
        out_specs=[
            pl.BlockSpec((8, 1), lambda i: (0, 0)),
            pl.BlockSpec((8, 1), lambda i: (0, 0)),
        ],
        out_shape=[
            jax.ShapeDtypeStruct((8, 1), jnp.float32),
            jax.ShapeDtypeStruct((8, 1), jnp.float32),
        ],
    )(bacc, attrs, Wm0, Wm1, Ws, Wp0, Wp1, batch2d)


def _sc_params():
    return pltpu.CompilerParams(needs_layout_passes=False,
                                use_tc_tiling_on_sc=False)


def _sc_mesh():
    return plsc.VectorSubcoreMesh(core_axis_name="c", subcore_axis_name="s")


def _unit_y1_16(px, py, pz, sv, dv):
    dx = plsc.load_gather(px, [dv]) - plsc.load_gather(px, [sv])
    dy = plsc.load_gather(py, [dv]) - plsc.load_gather(py, [sv])
    dz = plsc.load_gather(pz, [dv]) - plsc.load_gather(pz, [sv])
    r2 = dx * dx + dy * dy + dz * dz
    ii = plsc.bitcast(r2, jnp.int32)
    ii = 0x5F3759DF - jnp.right_shift(ii, 1)
    y = plsc.bitcast(ii, jnp.float32)
    for _ in range(3):
        y = y * (1.5 - 0.5 * r2 * y * y)
    r = r2 * y
    rc = _C1 / (r + 1e-9)
    return dx * rc, dy * rc, dz * rc


def _sc_y1(posx, posy, posz, edge_index):
    @functools.partial(
        pl.kernel,
        out_type=jax.ShapeDtypeStruct((3, _E), jnp.float32),
        mesh=_sc_mesh(),
        compiler_params=_sc_params(),
        scratch_types=[
            pltpu.VMEM((_NPAD,), jnp.float32),
            pltpu.VMEM((_NPAD,), jnp.float32),
            pltpu.VMEM((_NPAD,), jnp.float32),
            pltpu.VMEM((2, _K), jnp.int32),
            pltpu.VMEM((3, _K), jnp.float32),
        ],
    )
    def k(posx_h, posy_h, posz_h, ei_h, oy_h,
          px, py, pz, ei_v, y1_v):
        c = lax.axis_index("c")
        s = lax.axis_index("s")
        wid = c * _TILES + s
        pltpu.sync_copy(posx_h, px)
        pltpu.sync_copy(posy_h, py)
        pltpu.sync_copy(posz_h, pz)
        nchunks = (_NCHUNK - wid + 2 * _TILES - 1) // (2 * _TILES)

        def chunk(i, carry):
            e0 = (wid + 2 * _TILES * i) * _K
            pltpu.sync_copy(ei_h.at[:, pl.ds(e0, _K)], ei_v)
            for kk in range(8):
                sl = pl.ds(16 * kk, 16)
                yx, yy, yz = _unit_y1_16(px, py, pz, ei_v[0, sl], ei_v[1, sl])
                y1_v[0, sl] = yx
                y1_v[1, sl] = yy
                y1_v[2, sl] = yz
            pltpu.sync_copy(y1_v, oy_h.at[:, pl.ds(e0, _K)])
            return carry

        lax.fori_loop(0, nchunks, chunk, 0)

    return k(posx, posy, posz, edge_index)


def _sc_edge(y1, edge_index, nf4, w4):
    buf_t = [
        pltpu.VMEM((2, _K), jnp.int32),
        pltpu.VMEM((_K,), jnp.int32),
        pltpu.VMEM((_K, _CB), jnp.float32),
        pltpu.VMEM((_K, 64), jnp.float32),
        pltpu.VMEM((3, _K), jnp.float32),
    ]

    @functools.partial(
        pl.kernel,
        out_type=jax.ShapeDtypeStruct((_NBLK * _NPAD, _C), jnp.float32),
        mesh=_sc_mesh(),
        compiler_params=_sc_params(),
        scratch_types=buf_t + buf_t + [
            pltpu.VMEM((_K, _C), jnp.float32),
            pltpu.VMEM((_K // 2,), jnp.int32),
            pltpu.VMEM((_K // 2,), jnp.int32),
            pltpu.VMEM_SHARED((_NPAD, _C), jnp.float32),
            pltpu.SemaphoreType.DMA,
            pltpu.SemaphoreType.DMA,
            pltpu.SemaphoreType.DMA,
            pltpu.SemaphoreType.DMA,
            pltpu.SemaphoreType.DMA,
            pltpu.SemaphoreType.DMA,
        ],
    )
    def k(y1_h, ei_h, nf_h, w_h, out_h,
          ei_a, idx_a, fs_a, wv_a, y1_a,
          ei_b, idx_b, fs_b, wv_b, y1_b,
          out_v, dst0, dst1, acc, sem_a, sem_b, semg_a, semg_b,
          sem_s0, sem_s1):
        c = lax.axis_index("c")
        s = lax.axis_index("s")
        bufs_a = (ei_a, idx_a, fs_a, wv_a, y1_a, sem_a, semg_a)
        bufs_b = (ei_b, idx_b, fs_b, wv_b, y1_b, sem_b, semg_b)

        for p in range(2):
            b = 2 * c + p
            bn = b * _NPAD
            bE = b * _E

            def issue_inputs(i, bufs):
                ei_v, idx2, fs, wv, y1_v, sem, semg = bufs
                e0 = (s + _TILES * i) * _K
                pltpu.async_copy(ei_h.at[:, pl.ds(e0, _K)], ei_v, sem)
                pltpu.async_copy(w_h.at[pl.ds(bE + e0, _K), :], wv, sem)
                pltpu.async_copy(y1_h.at[:, pl.ds(e0, _K)], y1_v, sem)

            def wait_inputs_prep_gather(bufs):
                ei_v, idx2, fs, wv, y1_v, sem, semg = bufs
                pltpu.make_async_copy(ei_h.at[:, pl.ds(0, _K)], ei_v, sem).wait()
                pltpu.make_async_copy(w_h.at[pl.ds(0, _K), :], wv, sem).wait()
                pltpu.make_async_copy(y1_h.at[:, pl.ds(0, _K)], y1_v, sem).wait()
                for kk in range(8):
                    sl = pl.ds(16 * kk, 16)
                    idx2[sl] = ei_v[0, sl] + bn
                pltpu.async_copy(nf_h.at[idx2], fs, semg)

            halves = ((dst0, sem_s0, 0), (dst1, sem_s1, _K // 2))

            def process(bufs, i):
                ei_v, idx2, fs, wv, y1_v, sem, semg = bufs
                pltpu.make_async_copy(nf_h.at[idx2], fs, semg).wait()

                for dst_h, sem_h, j0 in halves:
                    hv = out_v.at[pl.ds(j0, _K // 2), :]

                    @pl.when(i > 0)
                    def _():
                        pltpu.make_async_copy(hv, acc.at[dst_h], sem_h).wait()
                    for kk in range(4):
                        dst_h[pl.ds(16 * kk, 16)] = ei_v[1, pl.ds(j0 + 16 * kk, 16)]

                    def edge(jj, carry2):
                        for u in range(2):
                            j = j0 + 2 * jj + u
                            f0 = fs[j, pl.ds(0, 16)]
                            f1 = fs[j, pl.ds(16, 16)]
                            w00 = wv[j, pl.ds(0, 16)]
                            w01 = wv[j, pl.ds(16, 16)]
                            w10 = wv[j, pl.ds(32, 16)]
                            w11 = wv[j, pl.ds(48, 16)]
                            jj16 = jnp.full((16,), j, jnp.int32)
                            z16 = jnp.zeros((16,), jnp.int32)
                            yx = plsc.load_gather(y1_v, [z16, jj16])
                            yy = plsc.load_gather(y1_v, [z16 + 1, jj16])
                            yz = plsc.load_gather(y1_v, [z16 + 2, jj16])
                            t0 = f0 * w10
                            t1 = f1 * w11
                            out_v[j, pl.ds(0, 16)] = f0 * w00
                            out_v[j, pl.ds(16, 16)] = f1 * w01
                            out_v[j, pl.ds(32, 16)] = t0 * yx
                            out_v[j, pl.ds(48, 16)] = t1 * yx
                            out_v[j, pl.ds(64, 16)] = t0 * yy
                            out_v[j, pl.ds(80, 16)] = t1 * yy
                            out_v[j, pl.ds(96, 16)] = t0 * yz
                            out_v[j, pl.ds(112, 16)] = t1 * yz
                        return carry2

                    lax.fori_loop(0, _K // 4, edge, 0)
                    pltpu.async_copy(hv, acc.at[dst_h], sem_h, add=True)

            def zero_row(r, carry):
                for kk in range(8):
                    out_v[r, pl.ds(16 * kk, 16)] = jnp.zeros((16,), jnp.float32)
                return carry

            lax.fori_loop(0, _K, zero_row, 0)
            for t in range(5):
                row0 = s * _ROWS_PER_TILE + t * _K
                pltpu.sync_copy(out_v.at[pl.ds(0, _K), :],
                                acc.at[pl.ds(row0, _K), :])
            plsc.subcore_barrier()

            nchunks = (_NCHUNK - s + _TILES - 1) // _TILES

            issue_inputs(0, bufs_a)
            wait_inputs_prep_gather(bufs_a)
            issue_inputs(1, bufs_b)

            def body(i, cur, nxt):
                @pl.when(i + 1 < nchunks)
                def _():
                    wait_inputs_prep_gather(nxt)
                process(cur, i)

                @pl.when(i + 2 < nchunks)
                def _():
                    issue_inputs(i + 2, cur)

            def chunk(i, carry):
                @pl.when(lax.rem(i, 2) == 0)
                def _():
                    body(i, bufs_a, bufs_b)

                @pl.when(lax.rem(i, 2) == 1)
                def _():
                    body(i, bufs_b, bufs_a)

                return carry

            lax.fori_loop(0, nchunks, chunk, 0)
            pltpu.make_async_copy(out_v.at[pl.ds(0, _K // 2), :],
                                  acc.at[dst0], sem_s0).wait()
            pltpu.make_async_copy(out_v.at[pl.ds(_K // 2, _K // 2), :],
                                  acc.at[dst1], sem_s1).wait()
            plsc.subcore_barrier()
            for t in range(5):
                row0 = s * _ROWS_PER_TILE + t * _K
                pltpu.sync_copy(acc.at[pl.ds(row0, _K), :],
                                out_h.at[pl.ds(b * _NPAD + row0, _K), :])

    return k(y1, edge_index, nf4, w4)


def _perms():
    colperm = np.concatenate([
        np.concatenate([np.arange(32 * b, 32 * b + 32),
                        128 + np.arange(32 * b, 32 * b + 32)])
        for b in range(_NBLK)])
    rowperm = np.concatenate([
        np.arange(128),
        np.array([128 + 3 * cc + ax for ax in range(3) for cc in range(128)])])
    return colperm, rowperm


def kernel(x, pos, edge_attr, W1, b1, R1a, R1b, Wm0_1, Wm1_1, Ws1,
           W2, b2, R2a, R2b, Wm0_2, Wm1_2, Ws2, Wp0, Wp1, edge_index, batch):
    colperm, rowperm = _perms()
    edge_index = edge_index.astype(jnp.int32)
    pos_p = jnp.pad(pos, ((0, _NPAD - _N), (0, 0)))
    posx = pos_p[:, 0]
    posy = pos_p[:, 1]
    posz = pos_p[:, 2]

    y1 = _sc_y1(posx, posy, posz, edge_index)

    r1, r2 = _radial_tc2(edge_attr, R1a, R1b[:, colperm],
                         R2a, R2b[:, colperm])
    r1 = r1.reshape(_NBLK * _E, 64)
    r2 = r2.reshape(_NBLK * _E, 64)

    f1 = _feats_tc(x, W1, b1).reshape(_NBLK * _NPAD, _CB)
    bacc1 = _sc_edge(y1, edge_index, f1, r1)
    u1, f2 = _node_tc(bacc1.reshape(_NBLK, _NPAD, _C), x,
                      Wm0_1, Wm1_1, Ws1, W2[rowperm], b2)

    bacc2 = _sc_edge(y1, edge_index, f2.reshape(_NBLK * _NPAD, _CB), r2)
    sums, counts = _node_final_tc(bacc2.reshape(_NBLK, _NPAD, _C), u1,
                                  Wm0_2, Wm1_2, Ws2[rowperm], Wp0, Wp1,
                                  batch.reshape(_N, 1).astype(jnp.int32))
    return sums / jnp.maximum(counts, 1.0)
